# baseline (speedup 1.0000x reference)
import functools

import jax
import jax.numpy as jnp
from jax import lax
from jax.experimental import pallas as pl
from jax.experimental.pallas import tpu as pltpu
from jax.experimental.pallas import tpu_sc as plsc

f32 = jnp.float32
i32 = jnp.int32

N = 10000
NP = 10240
E = 320000
E2 = N + E
NW = 32
CHUNK = 128
CPW = 82
GH = CPW // 2
EPW = CPW * CHUNK
E2P = NW * EPW
F_IN = 128
HID = 160
HEADS = 5
W16 = 16
CA = 96
CB = 64
GG = 8
NCLS = 20
BM = 512
GRID = NP // BM
RPT = NP // 16


def _tc_dense1(x_ref, w_ref, as_ref, ad_ref, ha_ref, hb_ref, s_ref, d_ref):
    h = jnp.dot(x_ref[...], w_ref[...], preferred_element_type=f32)
    ha_ref[...] = h[:, :CA]
    hb_ref[...] = h[:, CA:]
    s_ref[...] = jnp.dot(h, as_ref[...], preferred_element_type=f32)
    d_ref[...] = jnp.dot(h, ad_ref[...], preferred_element_type=f32)


def _ln(o, g, be):
    m = jnp.mean(o, axis=-1, keepdims=True)
    v = jnp.mean((o - m) * (o - m), axis=-1, keepdims=True)
    return (o - m) * lax.rsqrt(v + 1e-5) * g + be


def _tc_mid(p0a_ref, p1a_ref, p0b_ref, p1b_ref, b_ref, g_ref, be_ref, w_ref,
            as_ref, ad_ref, ha_ref, hb_ref, s_ref, d_ref):
    o = jnp.concatenate([p0a_ref[...] + p1a_ref[...],
                         p0b_ref[...] + p1b_ref[...]], axis=-1) + b_ref[...]
    o = jnp.where(o > 0, o, 0.01 * o)
    o = _ln(o, g_ref[...], be_ref[...])
    h = jnp.dot(o, w_ref[...], preferred_element_type=f32)
    ha_ref[...] = h[:, :CA]
    hb_ref[...] = h[:, CA:]
    s_ref[...] = jnp.dot(h, as_ref[...], preferred_element_type=f32)
    d_ref[...] = jnp.dot(h, ad_ref[...], preferred_element_type=f32)


def _tc_fin(p0a_ref, p1a_ref, p0b_ref, p1b_ref, b_ref, g_ref, be_ref, wm_ref,
            bm_ref, bt_ref, xo_ref, ps_ref, ct_ref):
    i = pl.program_id(0)
    o = jnp.concatenate([p0a_ref[...] + p1a_ref[...],
                         p0b_ref[...] + p1b_ref[...]], axis=-1) + b_ref[...]
    o = jnp.where(o > 0, o, 0.01 * o)
    o = _ln(o, g_ref[...], be_ref[...])
    xo = jnp.dot(o, wm_ref[...], preferred_element_type=f32) + bm_ref[...]
    xo = jnp.where(xo > 0, xo, 0.01 * xo)
    xo_ref[...] = xo
    bt = bt_ref[0, 0, :]
    rows = lax.broadcasted_iota(i32, (GG, BM), 0)
    msk = (rows == bt[None, :]).astype(f32)

    @pl.when(i == 0)
    def _():
        ps_ref[...] = jnp.zeros_like(ps_ref)
        ct_ref[...] = jnp.zeros_like(ct_ref)

    ps_ref[...] += jnp.dot(msk, xo, preferred_element_type=f32)
    ct_ref[...] += jnp.dot(msk, jnp.ones((BM, 128), f32),
                           preferred_element_type=f32)


def _tc_head(ps_ref, ct_ref, w1_ref, b1_ref, g_ref, be_ref, w2_ref, b2_ref,
             rec_ref):
    cnt = ct_ref[:, 0:1]
    pooled = ps_ref[...] / jnp.maximum(cnt, 1.0)
    r = jnp.dot(pooled, w1_ref[...], preferred_element_type=f32) + b1_ref[...]
    r = _ln(r, g_ref[...], be_ref[...])
    r = jnp.maximum(r, 0.0)
    rec_ref[...] = jnp.dot(r, w2_ref[...], preferred_element_type=f32) + b2_ref[...]


def _sc_pass1(as_hbm, ad_hbm, src_hbm, dst_hbm, z16_hbm, ex_hbm, den_hbm,
              src_v, dst_v,
              as0, ad0, ex0, as1, ad1, ex1, den_sh,
              sa0, sd0, se0, sc0, sa1, sd1, se1, sc1):
    cid = lax.axis_index("c")
    sid = lax.axis_index("s")
    wid = sid * 2 + cid
    row0 = sid * RPT
    pltpu.sync_copy(z16_hbm.at[pl.ds(row0, RPT)], den_sh.at[pl.ds(row0, RPT)])
    plsc.subcore_barrier()
    pltpu.sync_copy(src_hbm.at[wid], src_v)
    pltpu.sync_copy(dst_hbm.at[wid], dst_v)

    def issue(j, asb, adb, sa, sd):
        pltpu.async_copy(as_hbm.at[src_v.at[j]], asb, sa)
        pltpu.async_copy(ad_hbm.at[dst_v.at[j]], adb, sd)

    def wait_in(asb, adb, sa, sd):
        pltpu.make_async_copy(as_hbm.at[src_v.at[0]], asb, sa).wait()
        pltpu.make_async_copy(ad_hbm.at[dst_v.at[0]], adb, sd).wait()

    def wait_out(exb, se, sc):
        pltpu.make_async_copy(exb, ex_hbm.at[pl.ds(0, CHUNK)], se).wait()
        pltpu.make_async_copy(exb, den_sh.at[dst_v.at[0]], sc).wait()

    def exp_rows(asb, adb, exb):
        @plsc.parallel_loop(0, CHUNK, unroll=8)
        def _ew(i):
            a = asb[i, :] + adb[i, :]
            a = jnp.where(a > 0, a, 0.2 * a)
            exb[i, :] = jnp.exp(a)

    def half(g, j, asb, adb, exb, sa, sd, se, sc):
        wait_in(asb, adb, sa, sd)

        @pl.when(g > 0)
        def _():
            wait_out(exb, se, sc)

        exp_rows(asb, adb, exb)
        base = wid * EPW + j * CHUNK
        pltpu.async_copy(exb, ex_hbm.at[pl.ds(base, CHUNK)], se)
        pltpu.async_copy(exb, den_sh.at[dst_v.at[j]], sc, add=True)

    issue(0, as0, ad0, sa0, sd0)

    def gbody(g, carry):
        j0 = 2 * g
        issue(j0 + 1, as1, ad1, sa1, sd1)
        half(g, j0, as0, ad0, ex0, sa0, sd0, se0, sc0)

        @pl.when(g + 1 < GH)
        def _():
            issue(j0 + 2, as0, ad0, sa0, sd0)

        half(g, j0 + 1, as1, ad1, ex1, sa1, sd1, se1, sc1)
        return carry

    lax.fori_loop(0, GH, gbody, 0)
    wait_out(ex0, se0, sc0)
    wait_out(ex1, se1, sc1)
    plsc.subcore_barrier()
    pltpu.sync_copy(den_sh.at[pl.ds(row0, RPT)],
                    den_hbm.at[cid, pl.ds(row0, RPT)])


def _weight_rows(rows_v, att_v, h_first, h_last, col0):
    @plsc.parallel_loop(0, CHUNK, unroll=4)
    def _we(e):
        for hh in range(h_first, h_last + 1):
            av = plsc.load_gather(
                att_v, [jnp.full((16,), e, i32), jnp.full((16,), hh, i32)])
            for half in range(2):
                off = hh * 32 + half * 16 - col0
                r = rows_v[e, pl.ds(off, 16)]
                rows_v[e, pl.ds(off, 16)] = r * av


def _sc_pass2a(ex_hbm, d0_hbm, d1_hbm, h_hbm, src_hbm, dst_hbm, zc_hbm,
               att_hbm, outp_hbm,
               src_v, dst_v,
               ex0, d00, d10, rows0, ex1, d01, d11, rows1, acc_sh,
               se0, s00, s10, sr0, sa0, sc0, se1, s01, s11, sr1, sa1, sc1):
    cid = lax.axis_index("c")
    sid = lax.axis_index("s")
    wid = sid * 2 + cid
    row0 = sid * RPT
    pltpu.sync_copy(zc_hbm.at[pl.ds(row0, RPT)], acc_sh.at[pl.ds(row0, RPT)])
    plsc.subcore_barrier()
    pltpu.sync_copy(src_hbm.at[wid], src_v)
    pltpu.sync_copy(dst_hbm.at[wid], dst_v)

    def issue(j, exb, d0b, d1b, rowsb, se, s0, s1, sr):
        base = wid * EPW + j * CHUNK
        pltpu.async_copy(ex_hbm.at[pl.ds(base, CHUNK)], exb, se)
        pltpu.async_copy(d0_hbm.at[dst_v.at[j]], d0b, s0)
        pltpu.async_copy(d1_hbm.at[dst_v.at[j]], d1b, s1)
        pltpu.async_copy(h_hbm.at[src_v.at[j]], rowsb, sr)

    def wait_in(exb, d0b, d1b, rowsb, se, s0, s1, sr):
        pltpu.make_async_copy(ex_hbm.at[pl.ds(0, CHUNK)], exb, se).wait()
        pltpu.make_async_copy(d0_hbm.at[dst_v.at[0]], d0b, s0).wait()
        pltpu.make_async_copy(d1_hbm.at[dst_v.at[0]], d1b, s1).wait()
        pltpu.make_async_copy(h_hbm.at[src_v.at[0]], rowsb, sr).wait()

    def wait_out(exb, rowsb, sa, sc):
        pltpu.make_async_copy(exb, att_hbm.at[pl.ds(0, CHUNK)], sa).wait()
        pltpu.make_async_copy(rowsb, acc_sh.at[dst_v.at[0]], sc).wait()

    def half(g, j, exb, d0b, d1b, rowsb, se, s0, s1, sr, sa, sc):
        wait_in(exb, d0b, d1b, rowsb, se, s0, s1, sr)

        @pl.when(g > 0)
        def _():
            wait_out(exb, rowsb, sa, sc)

        @plsc.parallel_loop(0, CHUNK, unroll=8)
        def _ew(i):
            exb[i, :] = exb[i, :] / (d0b[i, :] + d1b[i, :] + 1e-16)

        base = wid * EPW + j * CHUNK
        pltpu.async_copy(exb, att_hbm.at[pl.ds(base, CHUNK)], sa)
        _weight_rows(rowsb, exb, 0, 2, 0)
        pltpu.async_copy(rowsb, acc_sh.at[dst_v.at[j]], sc, add=True)

    issue(0, ex0, d00, d10, rows0, se0, s00, s10, sr0)

    def gbody(g, carry):
        j0 = 2 * g
        issue(j0 + 1, ex1, d01, d11, rows1, se1, s01, s11, sr1)
        half(g, j0, ex0, d00, d10, rows0, se0, s00, s10, sr0, sa0, sc0)

        @pl.when(g + 1 < GH)
        def _():
            issue(j0 + 2, ex0, d00, d10, rows0, se0, s00, s10, sr0)

        half(g, j0 + 1, ex1, d01, d11, rows1, se1, s01, s11, sr1, sa1, sc1)
        return carry

    lax.fori_loop(0, GH, gbody, 0)
    wait_out(ex0, rows0, sa0, sc0)
    wait_out(ex1, rows1, sa1, sc1)
    plsc.subcore_barrier()
    pltpu.sync_copy(acc_sh.at[pl.ds(row0, RPT)],
                    outp_hbm.at[cid, pl.ds(row0, RPT)])


def _sc_pass2b(att_hbm, h_hbm, src_hbm, dst_hbm, zc_hbm, outp_hbm,
               src_v, dst_v, att0, rows0, att1, rows1, acc_sh,
               se0, sr0, sc0, se1, sr1, sc1):
    cid = lax.axis_index("c")
    sid = lax.axis_index("s")
    wid = sid * 2 + cid
    row0 = sid * RPT
    pltpu.sync_copy(zc_hbm.at[pl.ds(row0, RPT)], acc_sh.at[pl.ds(row0, RPT)])
    plsc.subcore_barrier()
    pltpu.sync_copy(src_hbm.at[wid], src_v)
    pltpu.sync_copy(dst_hbm.at[wid], dst_v)

    def issue(j, attb, rowsb, se, sr):
        base = wid * EPW + j * CHUNK
        pltpu.async_copy(att_hbm.at[pl.ds(base, CHUNK)], attb, se)
        pltpu.async_copy(h_hbm.at[src_v.at[j]], rowsb, sr)

    def wait_in(attb, rowsb, se, sr):
        pltpu.make_async_copy(att_hbm.at[pl.ds(0, CHUNK)], attb, se).wait()
        pltpu.make_async_copy(h_hbm.at[src_v.at[0]], rowsb, sr).wait()

    def wait_out(rowsb, sc):
        pltpu.make_async_copy(rowsb, acc_sh.at[dst_v.at[0]], sc).wait()

    def half(g, j, attb, rowsb, se, sr, sc):
        wait_in(attb, rowsb, se, sr)

        @pl.when(g > 0)
        def _():
            wait_out(rowsb, sc)

        _weight_rows(rowsb, attb, 3, 4, CA)
        pltpu.async_copy(rowsb, acc_sh.at[dst_v.at[j]], sc, add=True)

    issue(0, att0, rows0, se0, sr0)

    def gbody(g, carry):
        j0 = 2 * g
        issue(j0 + 1, att1, rows1, se1, sr1)
        half(g, j0, att0, rows0, se0, sr0, sc0)

        @pl.when(g + 1 < GH)
        def _():
            issue(j0 + 2, att0, rows0, se0, sr0)

        half(g, j0 + 1, att1, rows1, se1, sr1, sc1)
        return carry

    lax.fori_loop(0, GH, gbody, 0)
    wait_out(rows0, sc0)
    wait_out(rows1, sc1)
    plsc.subcore_barrier()
    pltpu.sync_copy(acc_sh.at[pl.ds(row0, RPT)],
                    outp_hbm.at[cid, pl.ds(row0, RPT)])


def _row_spec(c):
    return pl.BlockSpec((BM, c), lambda i: (i, 0))


def _fix_spec(r, c):
    return pl.BlockSpec((r, c), lambda i: (0, 0))


def _sds(*shape):
    return jax.ShapeDtypeStruct(shape, f32)


@jax.jit
def kernel(x, edge_index, batch, W1, a_src1, a_dst1, b1, W2, a_src2, a_dst2,
           b2, g1, be1, g2, be2, Wm, bm, Wr1, br1, gr, ber, Wr2, br2):
    x_pad = jnp.zeros((NP, F_IN), f32).at[:N].set(x)
    loop = jnp.arange(N, dtype=i32)
    padi = jnp.full((E2P - E2,), N, i32)
    srcf = jnp.concatenate([edge_index[0].astype(i32), loop, padi]
                           ).reshape(NW, CPW, CHUNK)
    dstf = jnp.concatenate([edge_index[1].astype(i32), loop, padi]
                           ).reshape(NW, CPW, CHUNK)
    batch3 = jnp.concatenate([batch.astype(i32), jnp.full((NP - N,), GG, i32)]
                             ).reshape(GRID, 1, BM)

    sel = (jnp.repeat(jnp.arange(HEADS), HID // HEADS)[:, None]
           == jnp.arange(W16)[None, :]).astype(f32)

    def mk_a(a):
        return a.reshape(HID, 1) * sel

    z16 = jnp.zeros((NP, W16), f32)
    z96 = jnp.zeros((NP, CA), f32)
    z64 = jnp.zeros((NP, CB), f32)

    tc1 = pl.pallas_call(
        _tc_dense1, grid=(GRID,),
        in_specs=[_row_spec(F_IN), _fix_spec(F_IN, HID),
                  _fix_spec(HID, W16), _fix_spec(HID, W16)],
        out_specs=[_row_spec(CA), _row_spec(CB), _row_spec(W16),
                   _row_spec(W16)],
        out_shape=[_sds(NP, CA), _sds(NP, CB), _sds(NP, W16), _sds(NP, W16)],
    )
    tc2 = pl.pallas_call(
        _tc_mid, grid=(GRID,),
        in_specs=[_row_spec(CA), _row_spec(CA), _row_spec(CB), _row_spec(CB),
                  _fix_spec(1, HID), _fix_spec(1, HID), _fix_spec(1, HID),
                  _fix_spec(HID, HID), _fix_spec(HID, W16),
                  _fix_spec(HID, W16)],
        out_specs=[_row_spec(CA), _row_spec(CB), _row_spec(W16),
                   _row_spec(W16)],
        out_shape=[_sds(NP, CA), _sds(NP, CB), _sds(NP, W16), _sds(NP, W16)],
    )
    tc3 = pl.pallas_call(
        _tc_fin, grid=(GRID,),
        in_specs=[_row_spec(CA), _row_spec(CA), _row_spec(CB), _row_spec(CB),
                  _fix_spec(1, HID), _fix_spec(1, HID), _fix_spec(1, HID),
                  _fix_spec(HID, HID), _fix_spec(1, HID),
                  pl.BlockSpec((1, 1, BM), lambda i: (i, 0, 0))],
        out_specs=[_row_spec(HID), _fix_spec(GG, HID), _fix_spec(GG, 128)],
        out_shape=[_sds(NP, HID), _sds(GG, HID), _sds(GG, 128)],
    )
    tc4 = pl.pallas_call(
        _tc_head,
        out_shape=_sds(GG, NCLS),
    )

    mesh = plsc.VectorSubcoreMesh(core_axis_name="c", subcore_axis_name="s")
    sc_params = pltpu.CompilerParams(use_tc_tiling_on_sc=False,
                                     needs_layout_passes=False)
    sc1 = pl.kernel(
        _sc_pass1,
        out_type=(_sds(E2P, W16), _sds(2, NP, W16)),
        mesh=mesh,
        compiler_params=sc_params,
        scratch_types=[
            pltpu.VMEM((CPW, CHUNK), i32), pltpu.VMEM((CPW, CHUNK), i32),
            pltpu.VMEM((CHUNK, W16), f32), pltpu.VMEM((CHUNK, W16), f32),
            pltpu.VMEM((CHUNK, W16), f32), pltpu.VMEM((CHUNK, W16), f32),
            pltpu.VMEM((CHUNK, W16), f32), pltpu.VMEM((CHUNK, W16), f32),
            pltpu.VMEM_SHARED((NP, W16), f32),
        ] + [pltpu.SemaphoreType.DMA] * 8)
    sc2a = pl.kernel(
        _sc_pass2a,
        out_type=(_sds(E2P, W16), _sds(2, NP, CA)),
        mesh=mesh,
        compiler_params=sc_params,
        scratch_types=[
            pltpu.VMEM((CPW, CHUNK), i32), pltpu.VMEM((CPW, CHUNK), i32),
            pltpu.VMEM((CHUNK, W16), f32), pltpu.VMEM((CHUNK, W16), f32),
            pltpu.VMEM((CHUNK, W16), f32), pltpu.VMEM((CHUNK, CA), f32),
            pltpu.VMEM((CHUNK, W16), f32), pltpu.VMEM((CHUNK, W16), f32),
            pltpu.VMEM((CHUNK, W16), f32), pltpu.VMEM((CHUNK, CA), f32),
            pltpu.VMEM_SHARED((NP, CA), f32),
        ] + [pltpu.SemaphoreType.DMA] * 12)
    sc2b = pl.kernel(
        _sc_pass2b,
        out_type=_sds(2, NP, CB),
        mesh=mesh,
        compiler_params=sc_params,
        scratch_types=[
            pltpu.VMEM((CPW, CHUNK), i32), pltpu.VMEM((CPW, CHUNK), i32),
            pltpu.VMEM((CHUNK, W16), f32), pltpu.VMEM((CHUNK, CB), f32),
            pltpu.VMEM((CHUNK, W16), f32), pltpu.VMEM((CHUNK, CB), f32),
            pltpu.VMEM_SHARED((NP, CB), f32),
        ] + [pltpu.SemaphoreType.DMA] * 6)

    ha1, hb1, as1, ad1 = tc1(x_pad, W1, mk_a(a_src1), mk_a(a_dst1))
    ex1, den1 = sc1(as1, ad1, srcf, dstf, z16)
    att1f, outa1 = sc2a(ex1, den1[0], den1[1], ha1, srcf, dstf, z96)
    outb1 = sc2b(att1f, hb1, srcf, dstf, z64)

    ha2, hb2, as2, ad2 = tc2(outa1[0], outa1[1], outb1[0], outb1[1],
                             b1.reshape(1, HID), g1.reshape(1, HID),
                             be1.reshape(1, HID), W2,
                             mk_a(a_src2), mk_a(a_dst2))
    ex2, den2 = sc1(as2, ad2, srcf, dstf, z16)
    att2f, outa2 = sc2a(ex2, den2[0], den2[1], ha2, srcf, dstf, z96)
    outb2 = sc2b(att2f, hb2, srcf, dstf, z64)

    xo_full, psum, cnt = tc3(outa2[0], outa2[1], outb2[0], outb2[1],
                             b2.reshape(1, HID), g2.reshape(1, HID),
                             be2.reshape(1, HID), Wm, bm.reshape(1, HID),
                             batch3)
    rec = tc4(psum, cnt, Wr1, br1.reshape(1, HID), gr.reshape(1, HID),
              ber.reshape(1, HID), Wr2, br2.reshape(1, NCLS))

    return (xo_full[:N], rec, att1f[:E2, :HEADS], att2f[:E2, :HEADS])


# speedup vs baseline: 1.0427x; 1.0427x over previous
import functools

import jax
import jax.numpy as jnp
from jax import lax
from jax.experimental import pallas as pl
from jax.experimental.pallas import tpu as pltpu
from jax.experimental.pallas import tpu_sc as plsc

f32 = jnp.float32
i32 = jnp.int32

N = 10000
NP = 10240
E = 320000
E2 = N + E
NW = 32
CHUNK = 128
CPW = 82
GH = CPW // 2
EPW = CPW * CHUNK
E2P = NW * EPW
F_IN = 128
HID = 160
HEADS = 5
W16 = 16
CA = 96
CB = 64
GG = 8
NCLS = 20
BM = 512
GRID = NP // BM
RPT = NP // 16


def _tc_dense1(x_ref, w_ref, as_ref, ad_ref, ha_ref, hb_ref, s_ref, d_ref):
    h = jnp.dot(x_ref[...], w_ref[...], preferred_element_type=f32)
    ha_ref[...] = h[:, :CA]
    hb_ref[...] = h[:, CA:]
    s_ref[...] = jnp.dot(h, as_ref[...], preferred_element_type=f32)
    d_ref[...] = jnp.dot(h, ad_ref[...], preferred_element_type=f32)


def _ln(o, g, be):
    m = jnp.mean(o, axis=-1, keepdims=True)
    v = jnp.mean((o - m) * (o - m), axis=-1, keepdims=True)
    return (o - m) * lax.rsqrt(v + 1e-5) * g + be


def _tc_mid(p0a_ref, p1a_ref, p0b_ref, p1b_ref, b_ref, g_ref, be_ref, w_ref,
            as_ref, ad_ref, ha_ref, hb_ref, s_ref, d_ref):
    o = jnp.concatenate([p0a_ref[...] + p1a_ref[...],
                         p0b_ref[...] + p1b_ref[...]], axis=-1) + b_ref[...]
    o = jnp.where(o > 0, o, 0.01 * o)
    o = _ln(o, g_ref[...], be_ref[...])
    h = jnp.dot(o, w_ref[...], preferred_element_type=f32)
    ha_ref[...] = h[:, :CA]
    hb_ref[...] = h[:, CA:]
    s_ref[...] = jnp.dot(h, as_ref[...], preferred_element_type=f32)
    d_ref[...] = jnp.dot(h, ad_ref[...], preferred_element_type=f32)


def _tc_fin(p0a_ref, p1a_ref, p0b_ref, p1b_ref, b_ref, g_ref, be_ref, wm_ref,
            bm_ref, bt_ref, xo_ref, ps_ref, ct_ref):
    i = pl.program_id(0)
    o = jnp.concatenate([p0a_ref[...] + p1a_ref[...],
                         p0b_ref[...] + p1b_ref[...]], axis=-1) + b_ref[...]
    o = jnp.where(o > 0, o, 0.01 * o)
    o = _ln(o, g_ref[...], be_ref[...])
    xo = jnp.dot(o, wm_ref[...], preferred_element_type=f32) + bm_ref[...]
    xo = jnp.where(xo > 0, xo, 0.01 * xo)
    xo_ref[...] = xo
    bt = bt_ref[0, 0, :]
    rows = lax.broadcasted_iota(i32, (GG, BM), 0)
    msk = (rows == bt[None, :]).astype(f32)

    @pl.when(i == 0)
    def _():
        ps_ref[...] = jnp.zeros_like(ps_ref)
        ct_ref[...] = jnp.zeros_like(ct_ref)

    ps_ref[...] += jnp.dot(msk, xo, preferred_element_type=f32)
    ct_ref[...] += jnp.dot(msk, jnp.ones((BM, 128), f32),
                           preferred_element_type=f32)


def _tc_head(ps_ref, ct_ref, w1_ref, b1_ref, g_ref, be_ref, w2_ref, b2_ref,
             rec_ref):
    cnt = ct_ref[:, 0:1]
    pooled = ps_ref[...] / jnp.maximum(cnt, 1.0)
    r = jnp.dot(pooled, w1_ref[...], preferred_element_type=f32) + b1_ref[...]
    r = _ln(r, g_ref[...], be_ref[...])
    r = jnp.maximum(r, 0.0)
    rec_ref[...] = jnp.dot(r, w2_ref[...], preferred_element_type=f32) + b2_ref[...]


def _zero_stripe(tmpb, acc_sh, row0, ncol):
    @plsc.parallel_loop(0, CHUNK, unroll=8)
    def _z(i):
        for k in range(ncol // 16):
            tmpb[i, pl.ds(k * 16, 16)] = jnp.zeros((16,), f32)

    for k in range(RPT // CHUNK):
        pltpu.sync_copy(tmpb, acc_sh.at[pl.ds(row0 + k * CHUNK, CHUNK)])


def _sc_pass1(as_hbm, ad_hbm, src_hbm, dst_hbm, ex_hbm, den_hbm,
              src_v, dst_v,
              as0, ad0, ex0, as1, ad1, ex1, den_sh,
              sa0, sd0, se0, sc0, sa1, sd1, se1, sc1):
    cid = lax.axis_index("c")
    sid = lax.axis_index("s")
    wid = sid * 2 + cid
    row0 = sid * RPT
    _zero_stripe(ex0, den_sh, row0, W16)
    plsc.subcore_barrier()
    pltpu.sync_copy(src_hbm.at[wid], src_v)
    pltpu.sync_copy(dst_hbm.at[wid], dst_v)

    def issue(j, asb, adb, sa, sd):
        pltpu.async_copy(as_hbm.at[src_v.at[j]], asb, sa)
        pltpu.async_copy(ad_hbm.at[dst_v.at[j]], adb, sd)

    def wait_in(asb, adb, sa, sd):
        pltpu.make_async_copy(as_hbm.at[src_v.at[0]], asb, sa).wait()
        pltpu.make_async_copy(ad_hbm.at[dst_v.at[0]], adb, sd).wait()

    def wait_out(exb, se, sc):
        pltpu.make_async_copy(exb, ex_hbm.at[pl.ds(0, CHUNK)], se).wait()
        pltpu.make_async_copy(exb, den_sh.at[dst_v.at[0]], sc).wait()

    def exp_rows(asb, adb, exb):
        @plsc.parallel_loop(0, CHUNK, unroll=8)
        def _ew(i):
            a = asb[i, :] + adb[i, :]
            a = jnp.where(a > 0, a, 0.2 * a)
            exb[i, :] = jnp.exp(a)

    def half(g, j, asb, adb, exb, sa, sd, se, sc):
        wait_in(asb, adb, sa, sd)

        @pl.when(g > 0)
        def _():
            wait_out(exb, se, sc)

        exp_rows(asb, adb, exb)
        base = wid * EPW + j * CHUNK
        pltpu.async_copy(exb, ex_hbm.at[pl.ds(base, CHUNK)], se)
        pltpu.async_copy(exb, den_sh.at[dst_v.at[j]], sc, add=True)

    issue(0, as0, ad0, sa0, sd0)

    def gbody(g, carry):
        j0 = 2 * g
        issue(j0 + 1, as1, ad1, sa1, sd1)
        half(g, j0, as0, ad0, ex0, sa0, sd0, se0, sc0)

        @pl.when(g + 1 < GH)
        def _():
            issue(j0 + 2, as0, ad0, sa0, sd0)

        half(g, j0 + 1, as1, ad1, ex1, sa1, sd1, se1, sc1)
        return carry

    lax.fori_loop(0, GH, gbody, 0)
    wait_out(ex0, se0, sc0)
    wait_out(ex1, se1, sc1)
    plsc.subcore_barrier()
    pltpu.sync_copy(den_sh.at[pl.ds(row0, RPT)],
                    den_hbm.at[cid, pl.ds(row0, RPT)])


def _weight_rows(rows_v, att_v, h_first, h_last, col0):
    @plsc.parallel_loop(0, CHUNK, unroll=4)
    def _we(e):
        row = att_v[e, :]
        for hh in range(h_first, h_last + 1):
            av = jnp.full((16,), row[hh], f32)
            for half in range(2):
                off = hh * 32 + half * 16 - col0
                r = rows_v[e, pl.ds(off, 16)]
                rows_v[e, pl.ds(off, 16)] = r * av


def _sc_pass2a(ex_hbm, den_hbm, h_hbm, src_hbm, dst_hbm,
               att_hbm, outp_hbm,
               src_v, dst_v,
               ex0, d00, d10, rows0, ex1, d01, d11, rows1, acc_sh,
               se0, s00, s10, sr0, sa0, sc0, se1, s01, s11, sr1, sa1, sc1):
    cid = lax.axis_index("c")
    sid = lax.axis_index("s")
    wid = sid * 2 + cid
    row0 = sid * RPT
    _zero_stripe(rows0, acc_sh, row0, CA)
    plsc.subcore_barrier()
    pltpu.sync_copy(src_hbm.at[wid], src_v)
    pltpu.sync_copy(dst_hbm.at[wid], dst_v)

    def issue(j, exb, d0b, d1b, rowsb, se, s0, s1, sr):
        base = wid * EPW + j * CHUNK
        pltpu.async_copy(ex_hbm.at[pl.ds(base, CHUNK)], exb, se)
        pltpu.async_copy(den_hbm.at[0].at[dst_v.at[j]], d0b, s0)
        pltpu.async_copy(den_hbm.at[1].at[dst_v.at[j]], d1b, s1)
        pltpu.async_copy(h_hbm.at[src_v.at[j]], rowsb, sr)

    def wait_in(exb, d0b, d1b, rowsb, se, s0, s1, sr):
        pltpu.make_async_copy(ex_hbm.at[pl.ds(0, CHUNK)], exb, se).wait()
        pltpu.make_async_copy(den_hbm.at[0].at[dst_v.at[0]], d0b, s0).wait()
        pltpu.make_async_copy(den_hbm.at[1].at[dst_v.at[0]], d1b, s1).wait()
        pltpu.make_async_copy(h_hbm.at[src_v.at[0]], rowsb, sr).wait()

    def wait_out(exb, rowsb, sa, sc):
        pltpu.make_async_copy(exb, att_hbm.at[pl.ds(0, CHUNK)], sa).wait()
        pltpu.make_async_copy(rowsb, acc_sh.at[dst_v.at[0]], sc).wait()

    def half(g, j, exb, d0b, d1b, rowsb, se, s0, s1, sr, sa, sc):
        wait_in(exb, d0b, d1b, rowsb, se, s0, s1, sr)

        @pl.when(g > 0)
        def _():
            wait_out(exb, rowsb, sa, sc)

        @plsc.parallel_loop(0, CHUNK, unroll=8)
        def _ew(i):
            exb[i, :] = exb[i, :] / (d0b[i, :] + d1b[i, :] + 1e-16)

        base = wid * EPW + j * CHUNK
        pltpu.async_copy(exb, att_hbm.at[pl.ds(base, CHUNK)], sa)
        _weight_rows(rowsb, exb, 0, 2, 0)
        pltpu.async_copy(rowsb, acc_sh.at[dst_v.at[j]], sc, add=True)

    issue(0, ex0, d00, d10, rows0, se0, s00, s10, sr0)

    def gbody(g, carry):
        j0 = 2 * g
        issue(j0 + 1, ex1, d01, d11, rows1, se1, s01, s11, sr1)
        half(g, j0, ex0, d00, d10, rows0, se0, s00, s10, sr0, sa0, sc0)

        @pl.when(g + 1 < GH)
        def _():
            issue(j0 + 2, ex0, d00, d10, rows0, se0, s00, s10, sr0)

        half(g, j0 + 1, ex1, d01, d11, rows1, se1, s01, s11, sr1, sa1, sc1)
        return carry

    lax.fori_loop(0, GH, gbody, 0)
    wait_out(ex0, rows0, sa0, sc0)
    wait_out(ex1, rows1, sa1, sc1)
    plsc.subcore_barrier()
    pltpu.sync_copy(acc_sh.at[pl.ds(row0, RPT)],
                    outp_hbm.at[cid, pl.ds(row0, RPT)])


def _sc_pass2b(att_hbm, h_hbm, src_hbm, dst_hbm, outp_hbm,
               src_v, dst_v, att0, rows0, att1, rows1, acc_sh,
               se0, sr0, sc0, se1, sr1, sc1):
    cid = lax.axis_index("c")
    sid = lax.axis_index("s")
    wid = sid * 2 + cid
    row0 = sid * RPT
    _zero_stripe(rows0, acc_sh, row0, CB)
    plsc.subcore_barrier()
    pltpu.sync_copy(src_hbm.at[wid], src_v)
    pltpu.sync_copy(dst_hbm.at[wid], dst_v)

    def issue(j, attb, rowsb, se, sr):
        base = wid * EPW + j * CHUNK
        pltpu.async_copy(att_hbm.at[pl.ds(base, CHUNK)], attb, se)
        pltpu.async_copy(h_hbm.at[src_v.at[j]], rowsb, sr)

    def wait_in(attb, rowsb, se, sr):
        pltpu.make_async_copy(att_hbm.at[pl.ds(0, CHUNK)], attb, se).wait()
        pltpu.make_async_copy(h_hbm.at[src_v.at[0]], rowsb, sr).wait()

    def wait_out(rowsb, sc):
        pltpu.make_async_copy(rowsb, acc_sh.at[dst_v.at[0]], sc).wait()

    def half(g, j, attb, rowsb, se, sr, sc):
        wait_in(attb, rowsb, se, sr)

        @pl.when(g > 0)
        def _():
            wait_out(rowsb, sc)

        _weight_rows(rowsb, attb, 3, 4, CA)
        pltpu.async_copy(rowsb, acc_sh.at[dst_v.at[j]], sc, add=True)

    issue(0, att0, rows0, se0, sr0)

    def gbody(g, carry):
        j0 = 2 * g
        issue(j0 + 1, att1, rows1, se1, sr1)
        half(g, j0, att0, rows0, se0, sr0, sc0)

        @pl.when(g + 1 < GH)
        def _():
            issue(j0 + 2, att0, rows0, se0, sr0)

        half(g, j0 + 1, att1, rows1, se1, sr1, sc1)
        return carry

    lax.fori_loop(0, GH, gbody, 0)
    wait_out(rows0, sc0)
    wait_out(rows1, sc1)
    plsc.subcore_barrier()
    pltpu.sync_copy(acc_sh.at[pl.ds(row0, RPT)],
                    outp_hbm.at[cid, pl.ds(row0, RPT)])


def _row_spec(c):
    return pl.BlockSpec((BM, c), lambda i: (i, 0))


def _fix_spec(r, c):
    return pl.BlockSpec((r, c), lambda i: (0, 0))


def _sds(*shape):
    return jax.ShapeDtypeStruct(shape, f32)


@jax.jit
def kernel(x, edge_index, batch, W1, a_src1, a_dst1, b1, W2, a_src2, a_dst2,
           b2, g1, be1, g2, be2, Wm, bm, Wr1, br1, gr, ber, Wr2, br2):
    x_pad = jnp.zeros((NP, F_IN), f32).at[:N].set(x)
    loop = jnp.arange(N, dtype=i32)
    padi = jnp.full((E2P - E2,), N, i32)
    srcf = jnp.concatenate([edge_index[0].astype(i32), loop, padi]
                           ).reshape(NW, CPW, CHUNK)
    dstf = jnp.concatenate([edge_index[1].astype(i32), loop, padi]
                           ).reshape(NW, CPW, CHUNK)
    batch3 = jnp.concatenate([batch.astype(i32), jnp.full((NP - N,), GG, i32)]
                             ).reshape(GRID, 1, BM)

    sel = (jnp.repeat(jnp.arange(HEADS), HID // HEADS)[:, None]
           == jnp.arange(W16)[None, :]).astype(f32)

    def mk_a(a):
        return a.reshape(HID, 1) * sel

    tc1 = pl.pallas_call(
        _tc_dense1, grid=(GRID,),
        in_specs=[_row_spec(F_IN), _fix_spec(F_IN, HID),
                  _fix_spec(HID, W16), _fix_spec(HID, W16)],
        out_specs=[_row_spec(CA), _row_spec(CB), _row_spec(W16),
                   _row_spec(W16)],
        out_shape=[_sds(NP, CA), _sds(NP, CB), _sds(NP, W16), _sds(NP, W16)],
    )
    tc2 = pl.pallas_call(
        _tc_mid, grid=(GRID,),
        in_specs=[_row_spec(CA), _row_spec(CA), _row_spec(CB), _row_spec(CB),
                  _fix_spec(1, HID), _fix_spec(1, HID), _fix_spec(1, HID),
                  _fix_spec(HID, HID), _fix_spec(HID, W16),
                  _fix_spec(HID, W16)],
        out_specs=[_row_spec(CA), _row_spec(CB), _row_spec(W16),
                   _row_spec(W16)],
        out_shape=[_sds(NP, CA), _sds(NP, CB), _sds(NP, W16), _sds(NP, W16)],
    )
    tc3 = pl.pallas_call(
        _tc_fin, grid=(GRID,),
        in_specs=[_row_spec(CA), _row_spec(CA), _row_spec(CB), _row_spec(CB),
                  _fix_spec(1, HID), _fix_spec(1, HID), _fix_spec(1, HID),
                  _fix_spec(HID, HID), _fix_spec(1, HID),
                  pl.BlockSpec((1, 1, BM), lambda i: (i, 0, 0))],
        out_specs=[_row_spec(HID), _fix_spec(GG, HID), _fix_spec(GG, 128)],
        out_shape=[_sds(NP, HID), _sds(GG, HID), _sds(GG, 128)],
    )
    tc4 = pl.pallas_call(
        _tc_head,
        out_shape=_sds(GG, NCLS),
    )

    mesh = plsc.VectorSubcoreMesh(core_axis_name="c", subcore_axis_name="s")
    sc_params = pltpu.CompilerParams(use_tc_tiling_on_sc=False,
                                     needs_layout_passes=False)
    sc1 = pl.kernel(
        _sc_pass1,
        out_type=(_sds(E2P, W16), _sds(2, NP, W16)),
        mesh=mesh,
        compiler_params=sc_params,
        scratch_types=[
            pltpu.VMEM((CPW, CHUNK), i32), pltpu.VMEM((CPW, CHUNK), i32),
            pltpu.VMEM((CHUNK, W16), f32), pltpu.VMEM((CHUNK, W16), f32),
            pltpu.VMEM((CHUNK, W16), f32), pltpu.VMEM((CHUNK, W16), f32),
            pltpu.VMEM((CHUNK, W16), f32), pltpu.VMEM((CHUNK, W16), f32),
            pltpu.VMEM_SHARED((NP, W16), f32),
        ] + [pltpu.SemaphoreType.DMA] * 8)
    sc2a = pl.kernel(
        _sc_pass2a,
        out_type=(_sds(E2P, W16), _sds(2, NP, CA)),
        mesh=mesh,
        compiler_params=sc_params,
        scratch_types=[
            pltpu.VMEM((CPW, CHUNK), i32), pltpu.VMEM((CPW, CHUNK), i32),
            pltpu.VMEM((CHUNK, W16), f32), pltpu.VMEM((CHUNK, W16), f32),
            pltpu.VMEM((CHUNK, W16), f32), pltpu.VMEM((CHUNK, CA), f32),
            pltpu.VMEM((CHUNK, W16), f32), pltpu.VMEM((CHUNK, W16), f32),
            pltpu.VMEM((CHUNK, W16), f32), pltpu.VMEM((CHUNK, CA), f32),
            pltpu.VMEM_SHARED((NP, CA), f32),
        ] + [pltpu.SemaphoreType.DMA] * 12)
    sc2b = pl.kernel(
        _sc_pass2b,
        out_type=_sds(2, NP, CB),
        mesh=mesh,
        compiler_params=sc_params,
        scratch_types=[
            pltpu.VMEM((CPW, CHUNK), i32), pltpu.VMEM((CPW, CHUNK), i32),
            pltpu.VMEM((CHUNK, W16), f32), pltpu.VMEM((CHUNK, CB), f32),
            pltpu.VMEM((CHUNK, W16), f32), pltpu.VMEM((CHUNK, CB), f32),
            pltpu.VMEM_SHARED((NP, CB), f32),
        ] + [pltpu.SemaphoreType.DMA] * 6)

    ha1, hb1, as1, ad1 = tc1(x_pad, W1, mk_a(a_src1), mk_a(a_dst1))
    ex1, den1 = sc1(as1, ad1, srcf, dstf)
    att1f, outa1 = sc2a(ex1, den1, ha1, srcf, dstf)
    outb1 = sc2b(att1f, hb1, srcf, dstf)

    ha2, hb2, as2, ad2 = tc2(outa1[0], outa1[1], outb1[0], outb1[1],
                             b1.reshape(1, HID), g1.reshape(1, HID),
                             be1.reshape(1, HID), W2,
                             mk_a(a_src2), mk_a(a_dst2))
    ex2, den2 = sc1(as2, ad2, srcf, dstf)
    att2f, outa2 = sc2a(ex2, den2, ha2, srcf, dstf)
    outb2 = sc2b(att2f, hb2, srcf, dstf)

    xo_full, psum, cnt = tc3(outa2[0], outa2[1], outb2[0], outb2[1],
                             b2.reshape(1, HID), g2.reshape(1, HID),
                             be2.reshape(1, HID), Wm, bm.reshape(1, HID),
                             batch3)
    rec = tc4(psum, cnt, Wr1, br1.reshape(1, HID), gr.reshape(1, HID),
              ber.reshape(1, HID), Wr2, br2.reshape(1, NCLS))

    return (xo_full[:N], rec, att1f[:E2, :HEADS], att2f[:E2, :HEADS])


# speedup vs baseline: 1.0435x; 1.0008x over previous
import functools

import jax
import jax.numpy as jnp
from jax import lax
from jax.experimental import pallas as pl
from jax.experimental.pallas import tpu as pltpu
from jax.experimental.pallas import tpu_sc as plsc

f32 = jnp.float32
i32 = jnp.int32

N = 10000
NP = 10240
E = 320000
E2 = N + E
NW = 32
CHUNK = 128
CPW = 82
GH = CPW // 2
EPW = CPW * CHUNK
E2P = NW * EPW
F_IN = 128
HID = 160
HEADS = 5
W16 = 16
CA = 96
CB = 64
GG = 8
NCLS = 20
BM = 512
GRID = NP // BM
RPT = NP // 16


def _tc_dense1(x_ref, w_ref, as_ref, ad_ref, ha_ref, hb_ref, s_ref, d_ref):
    h = jnp.dot(x_ref[...], w_ref[...], preferred_element_type=f32)
    ha_ref[...] = h[:, :CA]
    hb_ref[...] = h[:, CA:]
    s_ref[...] = jnp.dot(h, as_ref[...], preferred_element_type=f32)
    d_ref[...] = jnp.dot(h, ad_ref[...], preferred_element_type=f32)


def _ln(o, g, be):
    m = jnp.mean(o, axis=-1, keepdims=True)
    v = jnp.mean((o - m) * (o - m), axis=-1, keepdims=True)
    return (o - m) * lax.rsqrt(v + 1e-5) * g + be


def _tc_mid(p0a_ref, p1a_ref, p0b_ref, p1b_ref, b_ref, g_ref, be_ref, w_ref,
            as_ref, ad_ref, ha_ref, hb_ref, s_ref, d_ref):
    o = jnp.concatenate([p0a_ref[...] + p1a_ref[...],
                         p0b_ref[...] + p1b_ref[...]], axis=-1) + b_ref[...]
    o = jnp.where(o > 0, o, 0.01 * o)
    o = _ln(o, g_ref[...], be_ref[...])
    h = jnp.dot(o, w_ref[...], preferred_element_type=f32)
    ha_ref[...] = h[:, :CA]
    hb_ref[...] = h[:, CA:]
    s_ref[...] = jnp.dot(h, as_ref[...], preferred_element_type=f32)
    d_ref[...] = jnp.dot(h, ad_ref[...], preferred_element_type=f32)


def _tc_fin(p0a_ref, p1a_ref, p0b_ref, p1b_ref, b_ref, g_ref, be_ref, wm_ref,
            bm_ref, bt_ref, xo_ref, ps_ref, ct_ref):
    i = pl.program_id(0)
    o = jnp.concatenate([p0a_ref[...] + p1a_ref[...],
                         p0b_ref[...] + p1b_ref[...]], axis=-1) + b_ref[...]
    o = jnp.where(o > 0, o, 0.01 * o)
    o = _ln(o, g_ref[...], be_ref[...])
    xo = jnp.dot(o, wm_ref[...], preferred_element_type=f32) + bm_ref[...]
    xo = jnp.where(xo > 0, xo, 0.01 * xo)
    xo_ref[...] = xo
    bt = bt_ref[0, 0, :]
    rows = lax.broadcasted_iota(i32, (GG, BM), 0)
    msk = (rows == bt[None, :]).astype(f32)

    @pl.when(i == 0)
    def _():
        ps_ref[...] = jnp.zeros_like(ps_ref)
        ct_ref[...] = jnp.zeros_like(ct_ref)

    ps_ref[...] += jnp.dot(msk, xo, preferred_element_type=f32)
    ct_ref[...] += jnp.dot(msk, jnp.ones((BM, 128), f32),
                           preferred_element_type=f32)


def _tc_head(ps_ref, ct_ref, w1_ref, b1_ref, g_ref, be_ref, w2_ref, b2_ref,
             rec_ref):
    cnt = ct_ref[:, 0:1]
    pooled = ps_ref[...] / jnp.maximum(cnt, 1.0)
    r = jnp.dot(pooled, w1_ref[...], preferred_element_type=f32) + b1_ref[...]
    r = _ln(r, g_ref[...], be_ref[...])
    r = jnp.maximum(r, 0.0)
    rec_ref[...] = jnp.dot(r, w2_ref[...], preferred_element_type=f32) + b2_ref[...]


def _zero_stripe(tmpb, acc_sh, row0, ncol):
    @plsc.parallel_loop(0, CHUNK, unroll=8)
    def _z(i):
        for k in range(ncol // 16):
            tmpb[i, pl.ds(k * 16, 16)] = jnp.zeros((16,), f32)

    for k in range(RPT // CHUNK):
        pltpu.sync_copy(tmpb, acc_sh.at[pl.ds(row0 + k * CHUNK, CHUNK)])


def _sc_pass1(as_hbm, ad_hbm, src_hbm, dst_hbm, ex_hbm, den_hbm,
              src_v, dst_v,
              as0, ad0, ex0, as1, ad1, ex1, den_sh,
              sa0, sd0, se0, sc0, sa1, sd1, se1, sc1):
    cid = lax.axis_index("c")
    sid = lax.axis_index("s")
    wid = sid * 2 + cid
    row0 = sid * RPT
    _zero_stripe(ex0, den_sh, row0, W16)
    plsc.subcore_barrier()
    pltpu.sync_copy(src_hbm.at[wid], src_v)
    pltpu.sync_copy(dst_hbm.at[wid], dst_v)

    def issue(j, asb, adb, sa, sd):
        pltpu.async_copy(as_hbm.at[src_v.at[j]], asb, sa)
        pltpu.async_copy(ad_hbm.at[dst_v.at[j]], adb, sd)

    def wait_in(asb, adb, sa, sd):
        pltpu.make_async_copy(as_hbm.at[src_v.at[0]], asb, sa).wait()
        pltpu.make_async_copy(ad_hbm.at[dst_v.at[0]], adb, sd).wait()

    def wait_out(exb, se, sc):
        pltpu.make_async_copy(exb, ex_hbm.at[pl.ds(0, CHUNK)], se).wait()
        pltpu.make_async_copy(exb, den_sh.at[dst_v.at[0]], sc).wait()

    def exp_rows(asb, adb, exb):
        @plsc.parallel_loop(0, CHUNK, unroll=8)
        def _ew(i):
            a = asb[i, :] + adb[i, :]
            a = jnp.where(a > 0, a, 0.2 * a)
            exb[i, :] = jnp.exp(a)

    def half(g, j, asb, adb, exb, sa, sd, se, sc):
        wait_in(asb, adb, sa, sd)

        @pl.when(g > 0)
        def _():
            wait_out(exb, se, sc)

        exp_rows(asb, adb, exb)
        base = wid * EPW + j * CHUNK
        pltpu.async_copy(exb, ex_hbm.at[pl.ds(base, CHUNK)], se)
        pltpu.async_copy(exb, den_sh.at[dst_v.at[j]], sc, add=True)

    issue(0, as0, ad0, sa0, sd0)

    def gbody(g, carry):
        j0 = 2 * g
        issue(j0 + 1, as1, ad1, sa1, sd1)
        half(g, j0, as0, ad0, ex0, sa0, sd0, se0, sc0)

        @pl.when(g + 1 < GH)
        def _():
            issue(j0 + 2, as0, ad0, sa0, sd0)

        half(g, j0 + 1, as1, ad1, ex1, sa1, sd1, se1, sc1)
        return carry

    lax.fori_loop(0, GH, gbody, 0)
    wait_out(ex0, se0, sc0)
    wait_out(ex1, se1, sc1)
    plsc.subcore_barrier()
    pltpu.sync_copy(den_sh.at[pl.ds(row0, RPT)],
                    den_hbm.at[cid, pl.ds(row0, RPT)])


def _weight_rows(rows_v, att_v, h_first, h_last, col0):
    @plsc.parallel_loop(0, CHUNK, unroll=4)
    def _we(e):
        for hh in range(h_first, h_last + 1):
            av = plsc.load_gather(
                att_v, [jnp.full((16,), e, i32), jnp.full((16,), hh, i32)])
            for half in range(2):
                off = hh * 32 + half * 16 - col0
                r = rows_v[e, pl.ds(off, 16)]
                rows_v[e, pl.ds(off, 16)] = r * av


def _sc_pass2a(ex_hbm, d0_hbm, d1_hbm, h_hbm, src_hbm, dst_hbm,
               att_hbm, outp_hbm,
               src_v, dst_v,
               ex0, d00, d10, rows0, ex1, d01, d11, rows1, acc_sh,
               se0, s00, s10, sr0, sa0, sc0, se1, s01, s11, sr1, sa1, sc1):
    cid = lax.axis_index("c")
    sid = lax.axis_index("s")
    wid = sid * 2 + cid
    row0 = sid * RPT
    _zero_stripe(rows0, acc_sh, row0, CA)
    plsc.subcore_barrier()
    pltpu.sync_copy(src_hbm.at[wid], src_v)
    pltpu.sync_copy(dst_hbm.at[wid], dst_v)

    def issue(j, exb, d0b, d1b, rowsb, se, s0, s1, sr):
        base = wid * EPW + j * CHUNK
        pltpu.async_copy(ex_hbm.at[pl.ds(base, CHUNK)], exb, se)
        pltpu.async_copy(d0_hbm.at[dst_v.at[j]], d0b, s0)
        pltpu.async_copy(d1_hbm.at[dst_v.at[j]], d1b, s1)
        pltpu.async_copy(h_hbm.at[src_v.at[j]], rowsb, sr)

    def wait_in(exb, d0b, d1b, rowsb, se, s0, s1, sr):
        pltpu.make_async_copy(ex_hbm.at[pl.ds(0, CHUNK)], exb, se).wait()
        pltpu.make_async_copy(d0_hbm.at[dst_v.at[0]], d0b, s0).wait()
        pltpu.make_async_copy(d1_hbm.at[dst_v.at[0]], d1b, s1).wait()
        pltpu.make_async_copy(h_hbm.at[src_v.at[0]], rowsb, sr).wait()

    def wait_out(exb, rowsb, sa, sc):
        pltpu.make_async_copy(exb, att_hbm.at[pl.ds(0, CHUNK)], sa).wait()
        pltpu.make_async_copy(rowsb, acc_sh.at[dst_v.at[0]], sc).wait()

    def half(g, j, exb, d0b, d1b, rowsb, se, s0, s1, sr, sa, sc):
        wait_in(exb, d0b, d1b, rowsb, se, s0, s1, sr)

        @pl.when(g > 0)
        def _():
            wait_out(exb, rowsb, sa, sc)

        @plsc.parallel_loop(0, CHUNK, unroll=8)
        def _ew(i):
            exb[i, :] = exb[i, :] / (d0b[i, :] + d1b[i, :] + 1e-16)

        base = wid * EPW + j * CHUNK
        pltpu.async_copy(exb, att_hbm.at[pl.ds(base, CHUNK)], sa)
        _weight_rows(rowsb, exb, 0, 2, 0)
        pltpu.async_copy(rowsb, acc_sh.at[dst_v.at[j]], sc, add=True)

    issue(0, ex0, d00, d10, rows0, se0, s00, s10, sr0)

    def gbody(g, carry):
        j0 = 2 * g
        issue(j0 + 1, ex1, d01, d11, rows1, se1, s01, s11, sr1)
        half(g, j0, ex0, d00, d10, rows0, se0, s00, s10, sr0, sa0, sc0)

        @pl.when(g + 1 < GH)
        def _():
            issue(j0 + 2, ex0, d00, d10, rows0, se0, s00, s10, sr0)

        half(g, j0 + 1, ex1, d01, d11, rows1, se1, s01, s11, sr1, sa1, sc1)
        return carry

    lax.fori_loop(0, GH, gbody, 0)
    wait_out(ex0, rows0, sa0, sc0)
    wait_out(ex1, rows1, sa1, sc1)
    plsc.subcore_barrier()
    pltpu.sync_copy(acc_sh.at[pl.ds(row0, RPT)],
                    outp_hbm.at[cid, pl.ds(row0, RPT)])


def _sc_pass2b(att_hbm, h_hbm, src_hbm, dst_hbm, outp_hbm,
               src_v, dst_v, att0, rows0, att1, rows1, acc_sh,
               se0, sr0, sc0, se1, sr1, sc1):
    cid = lax.axis_index("c")
    sid = lax.axis_index("s")
    wid = sid * 2 + cid
    row0 = sid * RPT
    _zero_stripe(rows0, acc_sh, row0, CB)
    plsc.subcore_barrier()
    pltpu.sync_copy(src_hbm.at[wid], src_v)
    pltpu.sync_copy(dst_hbm.at[wid], dst_v)

    def issue(j, attb, rowsb, se, sr):
        base = wid * EPW + j * CHUNK
        pltpu.async_copy(att_hbm.at[pl.ds(base, CHUNK)], attb, se)
        pltpu.async_copy(h_hbm.at[src_v.at[j]], rowsb, sr)

    def wait_in(attb, rowsb, se, sr):
        pltpu.make_async_copy(att_hbm.at[pl.ds(0, CHUNK)], attb, se).wait()
        pltpu.make_async_copy(h_hbm.at[src_v.at[0]], rowsb, sr).wait()

    def wait_out(rowsb, sc):
        pltpu.make_async_copy(rowsb, acc_sh.at[dst_v.at[0]], sc).wait()

    def half(g, j, attb, rowsb, se, sr, sc):
        wait_in(attb, rowsb, se, sr)

        @pl.when(g > 0)
        def _():
            wait_out(rowsb, sc)

        _weight_rows(rowsb, attb, 3, 4, CA)
        pltpu.async_copy(rowsb, acc_sh.at[dst_v.at[j]], sc, add=True)

    issue(0, att0, rows0, se0, sr0)

    def gbody(g, carry):
        j0 = 2 * g
        issue(j0 + 1, att1, rows1, se1, sr1)
        half(g, j0, att0, rows0, se0, sr0, sc0)

        @pl.when(g + 1 < GH)
        def _():
            issue(j0 + 2, att0, rows0, se0, sr0)

        half(g, j0 + 1, att1, rows1, se1, sr1, sc1)
        return carry

    lax.fori_loop(0, GH, gbody, 0)
    wait_out(rows0, sc0)
    wait_out(rows1, sc1)
    plsc.subcore_barrier()
    pltpu.sync_copy(acc_sh.at[pl.ds(row0, RPT)],
                    outp_hbm.at[cid, pl.ds(row0, RPT)])


def _row_spec(c):
    return pl.BlockSpec((BM, c), lambda i: (i, 0))


def _fix_spec(r, c):
    return pl.BlockSpec((r, c), lambda i: (0, 0))


def _sds(*shape):
    return jax.ShapeDtypeStruct(shape, f32)


@jax.jit
def kernel(x, edge_index, batch, W1, a_src1, a_dst1, b1, W2, a_src2, a_dst2,
           b2, g1, be1, g2, be2, Wm, bm, Wr1, br1, gr, ber, Wr2, br2):
    x_pad = jnp.zeros((NP, F_IN), f32).at[:N].set(x)
    loop = jnp.arange(N, dtype=i32)
    padi = jnp.full((E2P - E2,), N, i32)
    srcf = jnp.concatenate([edge_index[0].astype(i32), loop, padi]
                           ).reshape(NW, CPW, CHUNK)
    dstf = jnp.concatenate([edge_index[1].astype(i32), loop, padi]
                           ).reshape(NW, CPW, CHUNK)
    batch3 = jnp.concatenate([batch.astype(i32), jnp.full((NP - N,), GG, i32)]
                             ).reshape(GRID, 1, BM)

    sel = (jnp.repeat(jnp.arange(HEADS), HID // HEADS)[:, None]
           == jnp.arange(W16)[None, :]).astype(f32)

    def mk_a(a):
        return a.reshape(HID, 1) * sel

    tc1 = pl.pallas_call(
        _tc_dense1, grid=(GRID,),
        in_specs=[_row_spec(F_IN), _fix_spec(F_IN, HID),
                  _fix_spec(HID, W16), _fix_spec(HID, W16)],
        out_specs=[_row_spec(CA), _row_spec(CB), _row_spec(W16),
                   _row_spec(W16)],
        out_shape=[_sds(NP, CA), _sds(NP, CB), _sds(NP, W16), _sds(NP, W16)],
    )
    tc2 = pl.pallas_call(
        _tc_mid, grid=(GRID,),
        in_specs=[_row_spec(CA), _row_spec(CA), _row_spec(CB), _row_spec(CB),
                  _fix_spec(1, HID), _fix_spec(1, HID), _fix_spec(1, HID),
                  _fix_spec(HID, HID), _fix_spec(HID, W16),
                  _fix_spec(HID, W16)],
        out_specs=[_row_spec(CA), _row_spec(CB), _row_spec(W16),
                   _row_spec(W16)],
        out_shape=[_sds(NP, CA), _sds(NP, CB), _sds(NP, W16), _sds(NP, W16)],
    )
    tc3 = pl.pallas_call(
        _tc_fin, grid=(GRID,),
        in_specs=[_row_spec(CA), _row_spec(CA), _row_spec(CB), _row_spec(CB),
                  _fix_spec(1, HID), _fix_spec(1, HID), _fix_spec(1, HID),
                  _fix_spec(HID, HID), _fix_spec(1, HID),
                  pl.BlockSpec((1, 1, BM), lambda i: (i, 0, 0))],
        out_specs=[_row_spec(HID), _fix_spec(GG, HID), _fix_spec(GG, 128)],
        out_shape=[_sds(NP, HID), _sds(GG, HID), _sds(GG, 128)],
    )
    tc4 = pl.pallas_call(
        _tc_head,
        out_shape=_sds(GG, NCLS),
    )

    mesh = plsc.VectorSubcoreMesh(core_axis_name="c", subcore_axis_name="s")
    sc_params = pltpu.CompilerParams(use_tc_tiling_on_sc=False,
                                     needs_layout_passes=False)
    sc1 = pl.kernel(
        _sc_pass1,
        out_type=(_sds(E2P, W16), _sds(2, NP, W16)),
        mesh=mesh,
        compiler_params=sc_params,
        scratch_types=[
            pltpu.VMEM((CPW, CHUNK), i32), pltpu.VMEM((CPW, CHUNK), i32),
            pltpu.VMEM((CHUNK, W16), f32), pltpu.VMEM((CHUNK, W16), f32),
            pltpu.VMEM((CHUNK, W16), f32), pltpu.VMEM((CHUNK, W16), f32),
            pltpu.VMEM((CHUNK, W16), f32), pltpu.VMEM((CHUNK, W16), f32),
            pltpu.VMEM_SHARED((NP, W16), f32),
        ] + [pltpu.SemaphoreType.DMA] * 8)
    sc2a = pl.kernel(
        _sc_pass2a,
        out_type=(_sds(E2P, W16), _sds(2, NP, CA)),
        mesh=mesh,
        compiler_params=sc_params,
        scratch_types=[
            pltpu.VMEM((CPW, CHUNK), i32), pltpu.VMEM((CPW, CHUNK), i32),
            pltpu.VMEM((CHUNK, W16), f32), pltpu.VMEM((CHUNK, W16), f32),
            pltpu.VMEM((CHUNK, W16), f32), pltpu.VMEM((CHUNK, CA), f32),
            pltpu.VMEM((CHUNK, W16), f32), pltpu.VMEM((CHUNK, W16), f32),
            pltpu.VMEM((CHUNK, W16), f32), pltpu.VMEM((CHUNK, CA), f32),
            pltpu.VMEM_SHARED((NP, CA), f32),
        ] + [pltpu.SemaphoreType.DMA] * 12)
    sc2b = pl.kernel(
        _sc_pass2b,
        out_type=_sds(2, NP, CB),
        mesh=mesh,
        compiler_params=sc_params,
        scratch_types=[
            pltpu.VMEM((CPW, CHUNK), i32), pltpu.VMEM((CPW, CHUNK), i32),
            pltpu.VMEM((CHUNK, W16), f32), pltpu.VMEM((CHUNK, CB), f32),
            pltpu.VMEM((CHUNK, W16), f32), pltpu.VMEM((CHUNK, CB), f32),
            pltpu.VMEM_SHARED((NP, CB), f32),
        ] + [pltpu.SemaphoreType.DMA] * 6)

    ha1, hb1, as1, ad1 = tc1(x_pad, W1, mk_a(a_src1), mk_a(a_dst1))
    ex1, den1 = sc1(as1, ad1, srcf, dstf)
    att1f, outa1 = sc2a(ex1, den1[0], den1[1], ha1, srcf, dstf)
    outb1 = sc2b(att1f, hb1, srcf, dstf)

    ha2, hb2, as2, ad2 = tc2(outa1[0], outa1[1], outb1[0], outb1[1],
                             b1.reshape(1, HID), g1.reshape(1, HID),
                             be1.reshape(1, HID), W2,
                             mk_a(a_src2), mk_a(a_dst2))
    ex2, den2 = sc1(as2, ad2, srcf, dstf)
    att2f, outa2 = sc2a(ex2, den2[0], den2[1], ha2, srcf, dstf)
    outb2 = sc2b(att2f, hb2, srcf, dstf)

    xo_full, psum, cnt = tc3(outa2[0], outa2[1], outb2[0], outb2[1],
                             b2.reshape(1, HID), g2.reshape(1, HID),
                             be2.reshape(1, HID), Wm, bm.reshape(1, HID),
                             batch3)
    rec = tc4(psum, cnt, Wr1, br1.reshape(1, HID), gr.reshape(1, HID),
              ber.reshape(1, HID), Wr2, br2.reshape(1, NCLS))

    return (xo_full[:N], rec, att1f[:E2, :HEADS], att2f[:E2, :HEADS])


# speedup vs baseline: 1.5917x; 1.5254x over previous
import functools

import jax
import jax.numpy as jnp
from jax import lax
from jax.experimental import pallas as pl
from jax.experimental.pallas import tpu as pltpu
from jax.experimental.pallas import tpu_sc as plsc

f32 = jnp.float32
i32 = jnp.int32

N = 10000
NP = 10240
E = 320000
E2 = N + E
NW = 32
CHUNK = 128
CPW = 82
GH = CPW // 2
EPW = CPW * CHUNK
E2P = NW * EPW
F_IN = 128
HID = 160
HEADS = 5
W16 = 16
CA = 96
CB = 64
GG = 8
NCLS = 20
BM = 512
GRID = NP // BM
RPT = NP // 16


def _tc_dense1(x_ref, w_ref, as_ref, ad_ref, ha_ref, hb_ref, s_ref, d_ref):
    h = jnp.dot(x_ref[...], w_ref[...], preferred_element_type=f32)
    ha_ref[...] = h[:, :CA]
    hb_ref[...] = h[:, CA:]
    s_ref[...] = jnp.dot(h, as_ref[...], preferred_element_type=f32)
    d_ref[...] = jnp.dot(h, ad_ref[...], preferred_element_type=f32)


def _ln(o, g, be):
    m = jnp.mean(o, axis=-1, keepdims=True)
    v = jnp.mean((o - m) * (o - m), axis=-1, keepdims=True)
    return (o - m) * lax.rsqrt(v + 1e-5) * g + be


def _tc_mid(p0a_ref, p1a_ref, p0b_ref, p1b_ref, b_ref, g_ref, be_ref, w_ref,
            as_ref, ad_ref, ha_ref, hb_ref, s_ref, d_ref):
    o = jnp.concatenate([p0a_ref[...] + p1a_ref[...],
                         p0b_ref[...] + p1b_ref[...]], axis=-1) + b_ref[...]
    o = jnp.where(o > 0, o, 0.01 * o)
    o = _ln(o, g_ref[...], be_ref[...])
    h = jnp.dot(o, w_ref[...], preferred_element_type=f32)
    ha_ref[...] = h[:, :CA]
    hb_ref[...] = h[:, CA:]
    s_ref[...] = jnp.dot(h, as_ref[...], preferred_element_type=f32)
    d_ref[...] = jnp.dot(h, ad_ref[...], preferred_element_type=f32)


def _tc_fin(p0a_ref, p1a_ref, p0b_ref, p1b_ref, b_ref, g_ref, be_ref, wm_ref,
            bm_ref, bt_ref, xo_ref, ps_ref, ct_ref):
    i = pl.program_id(0)
    o = jnp.concatenate([p0a_ref[...] + p1a_ref[...],
                         p0b_ref[...] + p1b_ref[...]], axis=-1) + b_ref[...]
    o = jnp.where(o > 0, o, 0.01 * o)
    o = _ln(o, g_ref[...], be_ref[...])
    xo = jnp.dot(o, wm_ref[...], preferred_element_type=f32) + bm_ref[...]
    xo = jnp.where(xo > 0, xo, 0.01 * xo)
    xo_ref[...] = xo
    bt = bt_ref[0, 0, :]
    rows = lax.broadcasted_iota(i32, (GG, BM), 0)
    msk = (rows == bt[None, :]).astype(f32)

    @pl.when(i == 0)
    def _():
        ps_ref[...] = jnp.zeros_like(ps_ref)
        ct_ref[...] = jnp.zeros_like(ct_ref)

    ps_ref[...] += jnp.dot(msk, xo, preferred_element_type=f32)
    ct_ref[...] += jnp.dot(msk, jnp.ones((BM, 128), f32),
                           preferred_element_type=f32)


def _tc_head(ps_ref, ct_ref, w1_ref, b1_ref, g_ref, be_ref, w2_ref, b2_ref,
             rec_ref):
    cnt = ct_ref[:, 0:1]
    pooled = ps_ref[...] / jnp.maximum(cnt, 1.0)
    r = jnp.dot(pooled, w1_ref[...], preferred_element_type=f32) + b1_ref[...]
    r = _ln(r, g_ref[...], be_ref[...])
    r = jnp.maximum(r, 0.0)
    rec_ref[...] = jnp.dot(r, w2_ref[...], preferred_element_type=f32) + b2_ref[...]


def _zero_stripe(tmpb, acc_sh, row0, ncol):
    @plsc.parallel_loop(0, CHUNK, unroll=8)
    def _z(i):
        for k in range(ncol // 16):
            tmpb[i, pl.ds(k * 16, 16)] = jnp.zeros((16,), f32)

    for k in range(RPT // CHUNK):
        pltpu.sync_copy(tmpb, acc_sh.at[pl.ds(row0 + k * CHUNK, CHUNK)])


def _sc_pass1(as_hbm, ad_hbm, src_hbm, dst_hbm, ex_hbm, den_hbm,
              src_v, dst_v,
              as0, ad0, ex0, as1, ad1, ex1, den_sh,
              sa0, sd0, se0, sc0, sa1, sd1, se1, sc1):
    cid = lax.axis_index("c")
    sid = lax.axis_index("s")
    wid = sid * 2 + cid
    row0 = sid * RPT
    _zero_stripe(ex0, den_sh, row0, W16)
    plsc.subcore_barrier()
    pltpu.sync_copy(src_hbm.at[wid], src_v)
    pltpu.sync_copy(dst_hbm.at[wid], dst_v)

    def issue(j, asb, adb, sa, sd):
        pltpu.async_copy(as_hbm.at[src_v.at[j]], asb, sa)
        pltpu.async_copy(ad_hbm.at[dst_v.at[j]], adb, sd)

    def wait_in(asb, adb, sa, sd):
        pltpu.make_async_copy(as_hbm.at[src_v.at[0]], asb, sa).wait()
        pltpu.make_async_copy(ad_hbm.at[dst_v.at[0]], adb, sd).wait()

    def wait_out(exb, se, sc):
        pltpu.make_async_copy(exb, ex_hbm.at[pl.ds(0, CHUNK)], se).wait()
        pltpu.make_async_copy(exb, den_sh.at[dst_v.at[0]], sc).wait()

    def exp_rows(asb, adb, exb):
        @plsc.parallel_loop(0, CHUNK, unroll=8)
        def _ew(i):
            a = asb[i, :] + adb[i, :]
            a = jnp.where(a > 0, a, 0.2 * a)
            exb[i, :] = jnp.exp(a)

    def half(g, j, asb, adb, exb, sa, sd, se, sc):
        wait_in(asb, adb, sa, sd)

        @pl.when(g > 0)
        def _():
            wait_out(exb, se, sc)

        exp_rows(asb, adb, exb)
        base = wid * EPW + j * CHUNK
        pltpu.async_copy(exb, ex_hbm.at[pl.ds(base, CHUNK)], se)
        pltpu.async_copy(exb, den_sh.at[dst_v.at[j]], sc, add=True)

    issue(0, as0, ad0, sa0, sd0)

    def gbody(g, carry):
        j0 = 2 * g
        issue(j0 + 1, as1, ad1, sa1, sd1)
        half(g, j0, as0, ad0, ex0, sa0, sd0, se0, sc0)

        @pl.when(g + 1 < GH)
        def _():
            issue(j0 + 2, as0, ad0, sa0, sd0)

        half(g, j0 + 1, as1, ad1, ex1, sa1, sd1, se1, sc1)
        return carry

    lax.fori_loop(0, GH, gbody, 0)
    wait_out(ex0, se0, sc0)
    wait_out(ex1, se1, sc1)
    plsc.subcore_barrier()
    pltpu.sync_copy(den_sh.at[pl.ds(row0, RPT)],
                    den_hbm.at[cid, pl.ds(row0, RPT)])


def _weight_rows(rows_v, att_v, h_first, h_last, col0):
    @plsc.parallel_loop(0, CHUNK, unroll=4)
    def _we(e):
        for hh in range(h_first, h_last + 1):
            av = plsc.load_gather(
                att_v, [jnp.full((16,), e, i32), jnp.full((16,), hh, i32)])
            for half in range(2):
                off = hh * 32 + half * 16 - col0
                r = rows_v[e, pl.ds(off, 16)]
                rows_v[e, pl.ds(off, 16)] = r * av


def _sc_pass2a(ex_hbm, d0_hbm, d1_hbm, h_hbm, src_hbm, dst_hbm,
               att_hbm, outp_hbm,
               src_v, dst_v,
               ex0, d00, d10, rows0, ex1, d01, d11, rows1, acc_sh,
               se0, s00, s10, sr0, sa0, sc0, se1, s01, s11, sr1, sa1, sc1):
    cid = lax.axis_index("c")
    sid = lax.axis_index("s")
    wid = sid * 2 + cid
    row0 = sid * RPT
    _zero_stripe(rows0, acc_sh, row0, CA)
    plsc.subcore_barrier()
    pltpu.sync_copy(src_hbm.at[wid], src_v)
    pltpu.sync_copy(dst_hbm.at[wid], dst_v)

    def issue(j, exb, d0b, d1b, rowsb, se, s0, s1, sr):
        base = wid * EPW + j * CHUNK
        pltpu.async_copy(ex_hbm.at[pl.ds(base, CHUNK)], exb, se)
        pltpu.async_copy(d0_hbm.at[dst_v.at[j]], d0b, s0)
        pltpu.async_copy(d1_hbm.at[dst_v.at[j]], d1b, s1)
        pltpu.async_copy(h_hbm.at[src_v.at[j]], rowsb, sr)

    def wait_in(exb, d0b, d1b, rowsb, se, s0, s1, sr):
        pltpu.make_async_copy(ex_hbm.at[pl.ds(0, CHUNK)], exb, se).wait()
        pltpu.make_async_copy(d0_hbm.at[dst_v.at[0]], d0b, s0).wait()
        pltpu.make_async_copy(d1_hbm.at[dst_v.at[0]], d1b, s1).wait()
        pltpu.make_async_copy(h_hbm.at[src_v.at[0]], rowsb, sr).wait()

    def wait_out(exb, rowsb, sa, sc):
        pltpu.make_async_copy(exb, att_hbm.at[pl.ds(0, CHUNK)], sa).wait()
        pltpu.make_async_copy(rowsb, acc_sh.at[dst_v.at[0]], sc).wait()

    def half(g, j, exb, d0b, d1b, rowsb, se, s0, s1, sr, sa, sc):
        wait_in(exb, d0b, d1b, rowsb, se, s0, s1, sr)

        @pl.when(g > 0)
        def _():
            wait_out(exb, rowsb, sa, sc)

        @plsc.parallel_loop(0, CHUNK, unroll=8)
        def _ew(i):
            exb[i, :] = exb[i, :] / (d0b[i, :] + d1b[i, :] + 1e-16)

        base = wid * EPW + j * CHUNK
        pltpu.async_copy(exb, att_hbm.at[pl.ds(base, CHUNK)], sa)
        _weight_rows(rowsb, exb, 0, 2, 0)
        pltpu.async_copy(rowsb, acc_sh.at[dst_v.at[j]], sc, add=True)

    issue(0, ex0, d00, d10, rows0, se0, s00, s10, sr0)

    def gbody(g, carry):
        j0 = 2 * g
        issue(j0 + 1, ex1, d01, d11, rows1, se1, s01, s11, sr1)
        half(g, j0, ex0, d00, d10, rows0, se0, s00, s10, sr0, sa0, sc0)

        @pl.when(g + 1 < GH)
        def _():
            issue(j0 + 2, ex0, d00, d10, rows0, se0, s00, s10, sr0)

        half(g, j0 + 1, ex1, d01, d11, rows1, se1, s01, s11, sr1, sa1, sc1)
        return carry

    lax.fori_loop(0, GH, gbody, 0)
    wait_out(ex0, rows0, sa0, sc0)
    wait_out(ex1, rows1, sa1, sc1)
    plsc.subcore_barrier()
    pltpu.sync_copy(acc_sh.at[pl.ds(row0, RPT)],
                    outp_hbm.at[cid, pl.ds(row0, RPT)])


def _sc_pass2b(att_hbm, h_hbm, src_hbm, dst_hbm, outp_hbm,
               src_v, dst_v, att0, rows0, att1, rows1, acc_sh,
               se0, sr0, sc0, se1, sr1, sc1):
    cid = lax.axis_index("c")
    sid = lax.axis_index("s")
    wid = sid * 2 + cid
    row0 = sid * RPT
    _zero_stripe(rows0, acc_sh, row0, CB)
    plsc.subcore_barrier()
    pltpu.sync_copy(src_hbm.at[wid], src_v)
    pltpu.sync_copy(dst_hbm.at[wid], dst_v)

    def issue(j, attb, rowsb, se, sr):
        base = wid * EPW + j * CHUNK
        pltpu.async_copy(att_hbm.at[pl.ds(base, CHUNK)], attb, se)
        pltpu.async_copy(h_hbm.at[src_v.at[j]], rowsb, sr)

    def wait_in(attb, rowsb, se, sr):
        pltpu.make_async_copy(att_hbm.at[pl.ds(0, CHUNK)], attb, se).wait()
        pltpu.make_async_copy(h_hbm.at[src_v.at[0]], rowsb, sr).wait()

    def wait_out(rowsb, sc):
        pltpu.make_async_copy(rowsb, acc_sh.at[dst_v.at[0]], sc).wait()

    def half(g, j, attb, rowsb, se, sr, sc):
        wait_in(attb, rowsb, se, sr)

        @pl.when(g > 0)
        def _():
            wait_out(rowsb, sc)

        _weight_rows(rowsb, attb, 3, 4, CA)
        pltpu.async_copy(rowsb, acc_sh.at[dst_v.at[j]], sc, add=True)

    issue(0, att0, rows0, se0, sr0)

    def gbody(g, carry):
        j0 = 2 * g
        issue(j0 + 1, att1, rows1, se1, sr1)
        half(g, j0, att0, rows0, se0, sr0, sc0)

        @pl.when(g + 1 < GH)
        def _():
            issue(j0 + 2, att0, rows0, se0, sr0)

        half(g, j0 + 1, att1, rows1, se1, sr1, sc1)
        return carry

    lax.fori_loop(0, GH, gbody, 0)
    wait_out(rows0, sc0)
    wait_out(rows1, sc1)
    plsc.subcore_barrier()
    pltpu.sync_copy(acc_sh.at[pl.ds(row0, RPT)],
                    outp_hbm.at[cid, pl.ds(row0, RPT)])


def _row_spec(c):
    return pl.BlockSpec((BM, c), lambda i: (i, 0))


def _fix_spec(r, c):
    return pl.BlockSpec((r, c), lambda i: (0, 0))


def _sds(*shape):
    return jax.ShapeDtypeStruct(shape, f32)


@jax.jit
def kernel(x, edge_index, batch, W1, a_src1, a_dst1, b1, W2, a_src2, a_dst2,
           b2, g1, be1, g2, be2, Wm, bm, Wr1, br1, gr, ber, Wr2, br2):
    x_pad = jnp.zeros((NP, F_IN), f32).at[:N].set(x)
    loop = jnp.arange(N, dtype=i32)
    padi = N + jnp.arange(E2P - E2, dtype=i32) % (NP - N)
    srcf = jnp.concatenate([edge_index[0].astype(i32), loop, padi]
                           ).reshape(NW, CPW, CHUNK)
    dstf = jnp.concatenate([edge_index[1].astype(i32), loop, padi]
                           ).reshape(NW, CPW, CHUNK)
    batch3 = jnp.concatenate([batch.astype(i32), jnp.full((NP - N,), GG, i32)]
                             ).reshape(GRID, 1, BM)

    sel = (jnp.repeat(jnp.arange(HEADS), HID // HEADS)[:, None]
           == jnp.arange(W16)[None, :]).astype(f32)

    def mk_a(a):
        return a.reshape(HID, 1) * sel

    tc1 = pl.pallas_call(
        _tc_dense1, grid=(GRID,),
        in_specs=[_row_spec(F_IN), _fix_spec(F_IN, HID),
                  _fix_spec(HID, W16), _fix_spec(HID, W16)],
        out_specs=[_row_spec(CA), _row_spec(CB), _row_spec(W16),
                   _row_spec(W16)],
        out_shape=[_sds(NP, CA), _sds(NP, CB), _sds(NP, W16), _sds(NP, W16)],
    )
    tc2 = pl.pallas_call(
        _tc_mid, grid=(GRID,),
        in_specs=[_row_spec(CA), _row_spec(CA), _row_spec(CB), _row_spec(CB),
                  _fix_spec(1, HID), _fix_spec(1, HID), _fix_spec(1, HID),
                  _fix_spec(HID, HID), _fix_spec(HID, W16),
                  _fix_spec(HID, W16)],
        out_specs=[_row_spec(CA), _row_spec(CB), _row_spec(W16),
                   _row_spec(W16)],
        out_shape=[_sds(NP, CA), _sds(NP, CB), _sds(NP, W16), _sds(NP, W16)],
    )
    tc3 = pl.pallas_call(
        _tc_fin, grid=(GRID,),
        in_specs=[_row_spec(CA), _row_spec(CA), _row_spec(CB), _row_spec(CB),
                  _fix_spec(1, HID), _fix_spec(1, HID), _fix_spec(1, HID),
                  _fix_spec(HID, HID), _fix_spec(1, HID),
                  pl.BlockSpec((1, 1, BM), lambda i: (i, 0, 0))],
        out_specs=[_row_spec(HID), _fix_spec(GG, HID), _fix_spec(GG, 128)],
        out_shape=[_sds(NP, HID), _sds(GG, HID), _sds(GG, 128)],
    )
    tc4 = pl.pallas_call(
        _tc_head,
        out_shape=_sds(GG, NCLS),
    )

    mesh = plsc.VectorSubcoreMesh(core_axis_name="c", subcore_axis_name="s")
    sc_params = pltpu.CompilerParams(use_tc_tiling_on_sc=False,
                                     needs_layout_passes=False)
    sc1 = pl.kernel(
        _sc_pass1,
        out_type=(_sds(E2P, W16), _sds(2, NP, W16)),
        mesh=mesh,
        compiler_params=sc_params,
        scratch_types=[
            pltpu.VMEM((CPW, CHUNK), i32), pltpu.VMEM((CPW, CHUNK), i32),
            pltpu.VMEM((CHUNK, W16), f32), pltpu.VMEM((CHUNK, W16), f32),
            pltpu.VMEM((CHUNK, W16), f32), pltpu.VMEM((CHUNK, W16), f32),
            pltpu.VMEM((CHUNK, W16), f32), pltpu.VMEM((CHUNK, W16), f32),
            pltpu.VMEM_SHARED((NP, W16), f32),
        ] + [pltpu.SemaphoreType.DMA] * 8)
    sc2a = pl.kernel(
        _sc_pass2a,
        out_type=(_sds(E2P, W16), _sds(2, NP, CA)),
        mesh=mesh,
        compiler_params=sc_params,
        scratch_types=[
            pltpu.VMEM((CPW, CHUNK), i32), pltpu.VMEM((CPW, CHUNK), i32),
            pltpu.VMEM((CHUNK, W16), f32), pltpu.VMEM((CHUNK, W16), f32),
            pltpu.VMEM((CHUNK, W16), f32), pltpu.VMEM((CHUNK, CA), f32),
            pltpu.VMEM((CHUNK, W16), f32), pltpu.VMEM((CHUNK, W16), f32),
            pltpu.VMEM((CHUNK, W16), f32), pltpu.VMEM((CHUNK, CA), f32),
            pltpu.VMEM_SHARED((NP, CA), f32),
        ] + [pltpu.SemaphoreType.DMA] * 12)
    sc2b = pl.kernel(
        _sc_pass2b,
        out_type=_sds(2, NP, CB),
        mesh=mesh,
        compiler_params=sc_params,
        scratch_types=[
            pltpu.VMEM((CPW, CHUNK), i32), pltpu.VMEM((CPW, CHUNK), i32),
            pltpu.VMEM((CHUNK, W16), f32), pltpu.VMEM((CHUNK, CB), f32),
            pltpu.VMEM((CHUNK, W16), f32), pltpu.VMEM((CHUNK, CB), f32),
            pltpu.VMEM_SHARED((NP, CB), f32),
        ] + [pltpu.SemaphoreType.DMA] * 6)

    ha1, hb1, as1, ad1 = tc1(x_pad, W1, mk_a(a_src1), mk_a(a_dst1))
    ex1, den1 = sc1(as1, ad1, srcf, dstf)
    att1f, outa1 = sc2a(ex1, den1[0], den1[1], ha1, srcf, dstf)
    outb1 = sc2b(att1f, hb1, srcf, dstf)

    ha2, hb2, as2, ad2 = tc2(outa1[0], outa1[1], outb1[0], outb1[1],
                             b1.reshape(1, HID), g1.reshape(1, HID),
                             be1.reshape(1, HID), W2,
                             mk_a(a_src2), mk_a(a_dst2))
    ex2, den2 = sc1(as2, ad2, srcf, dstf)
    att2f, outa2 = sc2a(ex2, den2[0], den2[1], ha2, srcf, dstf)
    outb2 = sc2b(att2f, hb2, srcf, dstf)

    xo_full, psum, cnt = tc3(outa2[0], outa2[1], outb2[0], outb2[1],
                             b2.reshape(1, HID), g2.reshape(1, HID),
                             be2.reshape(1, HID), Wm, bm.reshape(1, HID),
                             batch3)
    rec = tc4(psum, cnt, Wr1, br1.reshape(1, HID), gr.reshape(1, HID),
              ber.reshape(1, HID), Wr2, br2.reshape(1, NCLS))

    return (xo_full[:N], rec, att1f[:E2, :HEADS], att2f[:E2, :HEADS])


# speedup vs baseline: 1.5928x; 1.0007x over previous
import functools

import jax
import jax.numpy as jnp
from jax import lax
from jax.experimental import pallas as pl
from jax.experimental.pallas import tpu as pltpu
from jax.experimental.pallas import tpu_sc as plsc

f32 = jnp.float32
i32 = jnp.int32

N = 10000
NP = 10240
E = 320000
E2 = N + E
NW = 32
CHUNK = 128
CPW = 82
GH = CPW // 2
EPW = CPW * CHUNK
E2P = NW * EPW
F_IN = 128
HID = 160
HEADS = 5
W16 = 16
CA = 96
CB = 64
GG = 8
NCLS = 20
BM = 512
GRID = NP // BM
RPT = NP // 16


def _tc_dense1(x_ref, w_ref, as_ref, ad_ref, ha_ref, hb_ref, s_ref, d_ref):
    h = jnp.dot(x_ref[...], w_ref[...], preferred_element_type=f32)
    ha_ref[...] = h[:, :CA]
    hb_ref[...] = h[:, CA:]
    s_ref[...] = jnp.dot(h, as_ref[...], preferred_element_type=f32)
    d_ref[...] = jnp.dot(h, ad_ref[...], preferred_element_type=f32)


def _ln(o, g, be):
    m = jnp.mean(o, axis=-1, keepdims=True)
    v = jnp.mean((o - m) * (o - m), axis=-1, keepdims=True)
    return (o - m) * lax.rsqrt(v + 1e-5) * g + be


def _tc_mid(p0a_ref, p1a_ref, p0b_ref, p1b_ref, b_ref, g_ref, be_ref, w_ref,
            as_ref, ad_ref, ha_ref, hb_ref, s_ref, d_ref):
    o = jnp.concatenate([p0a_ref[...] + p1a_ref[...],
                         p0b_ref[...] + p1b_ref[...]], axis=-1) + b_ref[...]
    o = jnp.where(o > 0, o, 0.01 * o)
    o = _ln(o, g_ref[...], be_ref[...])
    h = jnp.dot(o, w_ref[...], preferred_element_type=f32)
    ha_ref[...] = h[:, :CA]
    hb_ref[...] = h[:, CA:]
    s_ref[...] = jnp.dot(h, as_ref[...], preferred_element_type=f32)
    d_ref[...] = jnp.dot(h, ad_ref[...], preferred_element_type=f32)


def _tc_fin(p0a_ref, p1a_ref, p0b_ref, p1b_ref, b_ref, g_ref, be_ref, wm_ref,
            bm_ref, bt_ref, xo_ref, ps_ref, ct_ref):
    i = pl.program_id(0)
    o = jnp.concatenate([p0a_ref[...] + p1a_ref[...],
                         p0b_ref[...] + p1b_ref[...]], axis=-1) + b_ref[...]
    o = jnp.where(o > 0, o, 0.01 * o)
    o = _ln(o, g_ref[...], be_ref[...])
    xo = jnp.dot(o, wm_ref[...], preferred_element_type=f32) + bm_ref[...]
    xo = jnp.where(xo > 0, xo, 0.01 * xo)
    xo_ref[...] = xo
    bt = bt_ref[0, 0, :]
    rows = lax.broadcasted_iota(i32, (GG, BM), 0)
    msk = (rows == bt[None, :]).astype(f32)

    @pl.when(i == 0)
    def _():
        ps_ref[...] = jnp.zeros_like(ps_ref)
        ct_ref[...] = jnp.zeros_like(ct_ref)

    ps_ref[...] += jnp.dot(msk, xo, preferred_element_type=f32)
    ct_ref[...] += jnp.dot(msk, jnp.ones((BM, 128), f32),
                           preferred_element_type=f32)


def _tc_head(ps_ref, ct_ref, w1_ref, b1_ref, g_ref, be_ref, w2_ref, b2_ref,
             rec_ref):
    cnt = ct_ref[:, 0:1]
    pooled = ps_ref[...] / jnp.maximum(cnt, 1.0)
    r = jnp.dot(pooled, w1_ref[...], preferred_element_type=f32) + b1_ref[...]
    r = _ln(r, g_ref[...], be_ref[...])
    r = jnp.maximum(r, 0.0)
    rec_ref[...] = jnp.dot(r, w2_ref[...], preferred_element_type=f32) + b2_ref[...]


def _zero_stripe(tmpb, acc_sh, row0, ncol):
    @plsc.parallel_loop(0, CHUNK, unroll=8)
    def _z(i):
        for k in range(ncol // 16):
            tmpb[i, pl.ds(k * 16, 16)] = jnp.zeros((16,), f32)

    for k in range(RPT // CHUNK):
        pltpu.sync_copy(tmpb, acc_sh.at[pl.ds(row0 + k * CHUNK, CHUNK)])


def _load_indices(pk_hbm, wid, pk_v, src_v, dst_v):
    pltpu.sync_copy(pk_hbm.at[wid], pk_v)

    @plsc.parallel_loop(0, CPW, unroll=2)
    def _up(j):
        for k in range(CHUNK // 16):
            sl = pl.ds(k * 16, 16)
            v = pk_v[j, sl]
            src_v[j, sl] = lax.shift_right_logical(v, 14)
            dst_v[j, sl] = v & 0x3FFF


def _sc_pass1(as_hbm, ad_hbm, pk_hbm, ex_hbm, den_hbm,
              pk_v, src_v, dst_v,
              as0, ad0, ex0, as1, ad1, ex1, den_sh,
              sa0, sd0, se0, sc0, sa1, sd1, se1, sc1):
    cid = lax.axis_index("c")
    sid = lax.axis_index("s")
    wid = sid * 2 + cid
    row0 = sid * RPT
    _zero_stripe(ex0, den_sh, row0, W16)
    plsc.subcore_barrier()
    _load_indices(pk_hbm, wid, pk_v, src_v, dst_v)

    def issue(j, asb, adb, sa, sd):
        pltpu.async_copy(as_hbm.at[src_v.at[j]], asb, sa)
        pltpu.async_copy(ad_hbm.at[dst_v.at[j]], adb, sd)

    def wait_in(asb, adb, sa, sd):
        pltpu.make_async_copy(as_hbm.at[src_v.at[0]], asb, sa).wait()
        pltpu.make_async_copy(ad_hbm.at[dst_v.at[0]], adb, sd).wait()

    def wait_out(exb, se, sc):
        pltpu.make_async_copy(exb, ex_hbm.at[pl.ds(0, CHUNK)], se).wait()
        pltpu.make_async_copy(exb, den_sh.at[dst_v.at[0]], sc).wait()

    def exp_rows(asb, adb, exb):
        @plsc.parallel_loop(0, CHUNK, unroll=8)
        def _ew(i):
            a = asb[i, :] + adb[i, :]
            a = jnp.where(a > 0, a, 0.2 * a)
            exb[i, :] = jnp.exp(a)

    def half(g, j, asb, adb, exb, sa, sd, se, sc):
        wait_in(asb, adb, sa, sd)

        @pl.when(g > 0)
        def _():
            wait_out(exb, se, sc)

        exp_rows(asb, adb, exb)
        base = wid * EPW + j * CHUNK
        pltpu.async_copy(exb, ex_hbm.at[pl.ds(base, CHUNK)], se)
        pltpu.async_copy(exb, den_sh.at[dst_v.at[j]], sc, add=True)

    issue(0, as0, ad0, sa0, sd0)

    def gbody(g, carry):
        j0 = 2 * g
        issue(j0 + 1, as1, ad1, sa1, sd1)
        half(g, j0, as0, ad0, ex0, sa0, sd0, se0, sc0)

        @pl.when(g + 1 < GH)
        def _():
            issue(j0 + 2, as0, ad0, sa0, sd0)

        half(g, j0 + 1, as1, ad1, ex1, sa1, sd1, se1, sc1)
        return carry

    lax.fori_loop(0, GH, gbody, 0)
    wait_out(ex0, se0, sc0)
    wait_out(ex1, se1, sc1)
    plsc.subcore_barrier()
    pltpu.sync_copy(den_sh.at[pl.ds(row0, RPT)],
                    den_hbm.at[cid, pl.ds(row0, RPT)])


def _weight_rows(rows_v, att_v, h_first, h_last, col0):
    @plsc.parallel_loop(0, CHUNK, unroll=8)
    def _we(e):
        for hh in range(h_first, h_last + 1):
            av = plsc.load_gather(
                att_v, [jnp.full((16,), e, i32), jnp.full((16,), hh, i32)])
            for half in range(2):
                off = hh * 32 + half * 16 - col0
                r = rows_v[e, pl.ds(off, 16)]
                rows_v[e, pl.ds(off, 16)] = r * av


def _sc_pass2a(ex_hbm, d0_hbm, d1_hbm, h_hbm, pk_hbm,
               att_hbm, outp_hbm,
               pk_v, src_v, dst_v,
               ex0, d00, d10, rows0, ex1, d01, d11, rows1, acc_sh,
               se0, s00, s10, sr0, sa0, sc0, se1, s01, s11, sr1, sa1, sc1):
    cid = lax.axis_index("c")
    sid = lax.axis_index("s")
    wid = sid * 2 + cid
    row0 = sid * RPT
    _zero_stripe(rows0, acc_sh, row0, CA)
    plsc.subcore_barrier()
    _load_indices(pk_hbm, wid, pk_v, src_v, dst_v)

    def issue(j, exb, d0b, d1b, rowsb, se, s0, s1, sr):
        base = wid * EPW + j * CHUNK
        pltpu.async_copy(ex_hbm.at[pl.ds(base, CHUNK)], exb, se)
        pltpu.async_copy(d0_hbm.at[dst_v.at[j]], d0b, s0)
        pltpu.async_copy(d1_hbm.at[dst_v.at[j]], d1b, s1)
        pltpu.async_copy(h_hbm.at[src_v.at[j]], rowsb, sr)

    def wait_in(exb, d0b, d1b, rowsb, se, s0, s1, sr):
        pltpu.make_async_copy(ex_hbm.at[pl.ds(0, CHUNK)], exb, se).wait()
        pltpu.make_async_copy(d0_hbm.at[dst_v.at[0]], d0b, s0).wait()
        pltpu.make_async_copy(d1_hbm.at[dst_v.at[0]], d1b, s1).wait()
        pltpu.make_async_copy(h_hbm.at[src_v.at[0]], rowsb, sr).wait()

    def wait_out(exb, rowsb, sa, sc):
        pltpu.make_async_copy(exb, att_hbm.at[pl.ds(0, CHUNK)], sa).wait()
        pltpu.make_async_copy(rowsb, acc_sh.at[dst_v.at[0]], sc).wait()

    def half(g, j, exb, d0b, d1b, rowsb, se, s0, s1, sr, sa, sc):
        wait_in(exb, d0b, d1b, rowsb, se, s0, s1, sr)

        @pl.when(g > 0)
        def _():
            wait_out(exb, rowsb, sa, sc)

        @plsc.parallel_loop(0, CHUNK, unroll=8)
        def _ew(i):
            exb[i, :] = exb[i, :] / (d0b[i, :] + d1b[i, :] + 1e-16)

        base = wid * EPW + j * CHUNK
        pltpu.async_copy(exb, att_hbm.at[pl.ds(base, CHUNK)], sa)
        _weight_rows(rowsb, exb, 0, 2, 0)
        pltpu.async_copy(rowsb, acc_sh.at[dst_v.at[j]], sc, add=True)

    issue(0, ex0, d00, d10, rows0, se0, s00, s10, sr0)

    def gbody(g, carry):
        j0 = 2 * g
        issue(j0 + 1, ex1, d01, d11, rows1, se1, s01, s11, sr1)
        half(g, j0, ex0, d00, d10, rows0, se0, s00, s10, sr0, sa0, sc0)

        @pl.when(g + 1 < GH)
        def _():
            issue(j0 + 2, ex0, d00, d10, rows0, se0, s00, s10, sr0)

        half(g, j0 + 1, ex1, d01, d11, rows1, se1, s01, s11, sr1, sa1, sc1)
        return carry

    lax.fori_loop(0, GH, gbody, 0)
    wait_out(ex0, rows0, sa0, sc0)
    wait_out(ex1, rows1, sa1, sc1)
    plsc.subcore_barrier()
    pltpu.sync_copy(acc_sh.at[pl.ds(row0, RPT)],
                    outp_hbm.at[cid, pl.ds(row0, RPT)])


def _sc_pass2b(att_hbm, h_hbm, pk_hbm, outp_hbm,
               pk_v, src_v, dst_v, att0, rows0, att1, rows1, acc_sh,
               se0, sr0, sc0, se1, sr1, sc1):
    cid = lax.axis_index("c")
    sid = lax.axis_index("s")
    wid = sid * 2 + cid
    row0 = sid * RPT
    _zero_stripe(rows0, acc_sh, row0, CB)
    plsc.subcore_barrier()
    _load_indices(pk_hbm, wid, pk_v, src_v, dst_v)

    def issue(j, attb, rowsb, se, sr):
        base = wid * EPW + j * CHUNK
        pltpu.async_copy(att_hbm.at[pl.ds(base, CHUNK)], attb, se)
        pltpu.async_copy(h_hbm.at[src_v.at[j]], rowsb, sr)

    def wait_in(attb, rowsb, se, sr):
        pltpu.make_async_copy(att_hbm.at[pl.ds(0, CHUNK)], attb, se).wait()
        pltpu.make_async_copy(h_hbm.at[src_v.at[0]], rowsb, sr).wait()

    def wait_out(rowsb, sc):
        pltpu.make_async_copy(rowsb, acc_sh.at[dst_v.at[0]], sc).wait()

    def half(g, j, attb, rowsb, se, sr, sc):
        wait_in(attb, rowsb, se, sr)

        @pl.when(g > 0)
        def _():
            wait_out(rowsb, sc)

        _weight_rows(rowsb, attb, 3, 4, CA)
        pltpu.async_copy(rowsb, acc_sh.at[dst_v.at[j]], sc, add=True)

    issue(0, att0, rows0, se0, sr0)

    def gbody(g, carry):
        j0 = 2 * g
        issue(j0 + 1, att1, rows1, se1, sr1)
        half(g, j0, att0, rows0, se0, sr0, sc0)

        @pl.when(g + 1 < GH)
        def _():
            issue(j0 + 2, att0, rows0, se0, sr0)

        half(g, j0 + 1, att1, rows1, se1, sr1, sc1)
        return carry

    lax.fori_loop(0, GH, gbody, 0)
    wait_out(rows0, sc0)
    wait_out(rows1, sc1)
    plsc.subcore_barrier()
    pltpu.sync_copy(acc_sh.at[pl.ds(row0, RPT)],
                    outp_hbm.at[cid, pl.ds(row0, RPT)])


def _row_spec(c):
    return pl.BlockSpec((BM, c), lambda i: (i, 0))


def _fix_spec(r, c):
    return pl.BlockSpec((r, c), lambda i: (0, 0))


def _sds(*shape):
    return jax.ShapeDtypeStruct(shape, f32)


@jax.jit
def kernel(x, edge_index, batch, W1, a_src1, a_dst1, b1, W2, a_src2, a_dst2,
           b2, g1, be1, g2, be2, Wm, bm, Wr1, br1, gr, ber, Wr2, br2):
    x_pad = jnp.zeros((NP, F_IN), f32).at[:N].set(x)
    loop = jnp.arange(N, dtype=i32)
    padi = N + jnp.arange(E2P - E2, dtype=i32) % (NP - N)
    srcf = jnp.concatenate([edge_index[0].astype(i32), loop, padi])
    dstf = jnp.concatenate([edge_index[1].astype(i32), loop, padi])
    pkf = ((srcf << 14) | dstf).reshape(NW, CPW, CHUNK)
    batch3 = jnp.concatenate([batch.astype(i32), jnp.full((NP - N,), GG, i32)]
                             ).reshape(GRID, 1, BM)

    sel = (jnp.repeat(jnp.arange(HEADS), HID // HEADS)[:, None]
           == jnp.arange(W16)[None, :]).astype(f32)

    def mk_a(a):
        return a.reshape(HID, 1) * sel

    tc1 = pl.pallas_call(
        _tc_dense1, grid=(GRID,),
        in_specs=[_row_spec(F_IN), _fix_spec(F_IN, HID),
                  _fix_spec(HID, W16), _fix_spec(HID, W16)],
        out_specs=[_row_spec(CA), _row_spec(CB), _row_spec(W16),
                   _row_spec(W16)],
        out_shape=[_sds(NP, CA), _sds(NP, CB), _sds(NP, W16), _sds(NP, W16)],
    )
    tc2 = pl.pallas_call(
        _tc_mid, grid=(GRID,),
        in_specs=[_row_spec(CA), _row_spec(CA), _row_spec(CB), _row_spec(CB),
                  _fix_spec(1, HID), _fix_spec(1, HID), _fix_spec(1, HID),
                  _fix_spec(HID, HID), _fix_spec(HID, W16),
                  _fix_spec(HID, W16)],
        out_specs=[_row_spec(CA), _row_spec(CB), _row_spec(W16),
                   _row_spec(W16)],
        out_shape=[_sds(NP, CA), _sds(NP, CB), _sds(NP, W16), _sds(NP, W16)],
    )
    tc3 = pl.pallas_call(
        _tc_fin, grid=(GRID,),
        in_specs=[_row_spec(CA), _row_spec(CA), _row_spec(CB), _row_spec(CB),
                  _fix_spec(1, HID), _fix_spec(1, HID), _fix_spec(1, HID),
                  _fix_spec(HID, HID), _fix_spec(1, HID),
                  pl.BlockSpec((1, 1, BM), lambda i: (i, 0, 0))],
        out_specs=[_row_spec(HID), _fix_spec(GG, HID), _fix_spec(GG, 128)],
        out_shape=[_sds(NP, HID), _sds(GG, HID), _sds(GG, 128)],
    )
    tc4 = pl.pallas_call(
        _tc_head,
        out_shape=_sds(GG, NCLS),
    )

    mesh = plsc.VectorSubcoreMesh(core_axis_name="c", subcore_axis_name="s")
    sc_params = pltpu.CompilerParams(use_tc_tiling_on_sc=False,
                                     needs_layout_passes=False)
    sc1 = pl.kernel(
        _sc_pass1,
        out_type=(_sds(E2P, W16), _sds(2, NP, W16)),
        mesh=mesh,
        compiler_params=sc_params,
        scratch_types=[
            pltpu.VMEM((CPW, CHUNK), i32), pltpu.VMEM((CPW, CHUNK), i32),
            pltpu.VMEM((CPW, CHUNK), i32),
            pltpu.VMEM((CHUNK, W16), f32), pltpu.VMEM((CHUNK, W16), f32),
            pltpu.VMEM((CHUNK, W16), f32), pltpu.VMEM((CHUNK, W16), f32),
            pltpu.VMEM((CHUNK, W16), f32), pltpu.VMEM((CHUNK, W16), f32),
            pltpu.VMEM_SHARED((NP, W16), f32),
        ] + [pltpu.SemaphoreType.DMA] * 8)
    sc2a = pl.kernel(
        _sc_pass2a,
        out_type=(_sds(E2P, W16), _sds(2, NP, CA)),
        mesh=mesh,
        compiler_params=sc_params,
        scratch_types=[
            pltpu.VMEM((CPW, CHUNK), i32), pltpu.VMEM((CPW, CHUNK), i32),
            pltpu.VMEM((CPW, CHUNK), i32),
            pltpu.VMEM((CHUNK, W16), f32), pltpu.VMEM((CHUNK, W16), f32),
            pltpu.VMEM((CHUNK, W16), f32), pltpu.VMEM((CHUNK, CA), f32),
            pltpu.VMEM((CHUNK, W16), f32), pltpu.VMEM((CHUNK, W16), f32),
            pltpu.VMEM((CHUNK, W16), f32), pltpu.VMEM((CHUNK, CA), f32),
            pltpu.VMEM_SHARED((NP, CA), f32),
        ] + [pltpu.SemaphoreType.DMA] * 12)
    sc2b = pl.kernel(
        _sc_pass2b,
        out_type=_sds(2, NP, CB),
        mesh=mesh,
        compiler_params=sc_params,
        scratch_types=[
            pltpu.VMEM((CPW, CHUNK), i32), pltpu.VMEM((CPW, CHUNK), i32),
            pltpu.VMEM((CPW, CHUNK), i32),
            pltpu.VMEM((CHUNK, W16), f32), pltpu.VMEM((CHUNK, CB), f32),
            pltpu.VMEM((CHUNK, W16), f32), pltpu.VMEM((CHUNK, CB), f32),
            pltpu.VMEM_SHARED((NP, CB), f32),
        ] + [pltpu.SemaphoreType.DMA] * 6)

    ha1, hb1, as1, ad1 = tc1(x_pad, W1, mk_a(a_src1), mk_a(a_dst1))
    ex1, den1 = sc1(as1, ad1, pkf)
    att1f, outa1 = sc2a(ex1, den1[0], den1[1], ha1, pkf)
    outb1 = sc2b(att1f, hb1, pkf)

    ha2, hb2, as2, ad2 = tc2(outa1[0], outa1[1], outb1[0], outb1[1],
                             b1.reshape(1, HID), g1.reshape(1, HID),
                             be1.reshape(1, HID), W2,
                             mk_a(a_src2), mk_a(a_dst2))
    ex2, den2 = sc1(as2, ad2, pkf)
    att2f, outa2 = sc2a(ex2, den2[0], den2[1], ha2, pkf)
    outb2 = sc2b(att2f, hb2, pkf)

    xo_full, psum, cnt = tc3(outa2[0], outa2[1], outb2[0], outb2[1],
                             b2.reshape(1, HID), g2.reshape(1, HID),
                             be2.reshape(1, HID), Wm, bm.reshape(1, HID),
                             batch3)
    rec = tc4(psum, cnt, Wr1, br1.reshape(1, HID), gr.reshape(1, HID),
              ber.reshape(1, HID), Wr2, br2.reshape(1, NCLS))

    return (xo_full[:N], rec, att1f[:E2, :HEADS], att2f[:E2, :HEADS])
